# baseline (speedup 1.0000x reference)
import functools

import jax
import jax.numpy as jnp
from jax import lax
from jax.experimental import pallas as pl
from jax.experimental.pallas import tpu as pltpu
from jax.experimental.pallas import tpu_sc as plsc

N = 8192 * 6
E = 8192 * 5
NC, NS = 2, 16
NW = NC * NS
EPW = E // NW
EPT = E // NS
NPT = N // NS
IB = 128
NRANGE = 2
NR = N // NRANGE
RPT = NR // NS

_SC_PARAMS = pltpu.CompilerParams(use_tc_tiling_on_sc=False)


@functools.cache
def _sc_kernels():
    mesh = plsc.VectorSubcoreMesh(
        core_axis_name="c", subcore_axis_name="s",
        num_cores=NC, num_subcores=NS)
    g0 = _make_sc_gather_deg(mesh)
    s2 = _make_sc_scatter_msg(mesh)
    s3 = _make_sc_gcn(mesh)
    return g0, s2, s3


def _make_sc_gather_deg(mesh):
    return functools.partial(
        pl.kernel,
        out_type=(jax.ShapeDtypeStruct((E, 128), jnp.float32),
                  jax.ShapeDtypeStruct((N, 128), jnp.float32)),
        mesh=mesh,
        scratch_types=(
            pltpu.VMEM((EPW // IB, IB), jnp.int32),
            pltpu.VMEM((EPT // IB, IB), jnp.int32),
            pltpu.VMEM((IB, 128), jnp.float32),
            pltpu.VMEM((IB, 8), jnp.float32),
            pltpu.VMEM_SHARED((N, 8), jnp.float32),
            pltpu.SemaphoreType.DMA,
        ),
        compiler_params=_SC_PARAMS,
    )(_sc_gather_deg_body)


def _sc_gather_deg_body(xpad_hbm, src3d_hbm, dst3d_hbm, ones_hbm, zeros8_hbm,
                        xsrc_hbm, deg_hbm,
                        sidx_v, didx_v, rows_v, ones_v, deg_sp, sem):
    cid = lax.axis_index("c")
    sid = lax.axis_index("s")
    wid = sid * NC + cid
    pltpu.sync_copy(src3d_hbm.at[wid], sidx_v)
    for j in range(EPW // IB):
        pltpu.async_copy(xpad_hbm.at[sidx_v.at[j]], rows_v, sem).wait()
        pltpu.sync_copy(rows_v, xsrc_hbm.at[pl.ds(wid * EPW + j * IB, IB)])

    @pl.when(cid == 0)
    def _deg():
        pltpu.sync_copy(zeros8_hbm, deg_sp.at[pl.ds(sid * NPT, NPT)])
        pltpu.sync_copy(dst3d_hbm.at[sid], didx_v)
        pltpu.sync_copy(ones_hbm, ones_v)
        plsc.subcore_barrier()
        for j in range(EPT // IB):
            pltpu.sync_copy(ones_v, deg_sp.at[didx_v.at[j]], add=True)
        plsc.subcore_barrier()
        pltpu.sync_copy(deg_sp.at[pl.ds(sid * NPT, NPT)],
                        deg_hbm.at[pl.ds(sid * NPT, NPT), pl.ds(0, 8)])


def _make_sc_scatter_msg(mesh):
    return functools.partial(
        pl.kernel,
        out_type=jax.ShapeDtypeStruct((N, 128), jnp.bfloat16),
        mesh=mesh,
        scratch_types=(
            pltpu.VMEM((EPT // IB, IB), jnp.int32),
            pltpu.VMEM((EPT // IB, IB), jnp.int32),
            pltpu.VMEM((IB, 128), jnp.bfloat16),
            pltpu.VMEM_SHARED((NR + 8, 128), jnp.bfloat16),
        ),
        compiler_params=_SC_PARAMS,
    )(_sc_scatter_msg_body)


def _rewrite_range_idx(didx_v, didx_r, lo):
    hi = lo + NR
    for r in range(EPT // IB):
        for k in range(IB // 16):
            v = didx_v[r, pl.ds(16 * k, 16)]
            ok = (v >= lo) & (v < hi)
            didx_r[r, pl.ds(16 * k, 16)] = jnp.where(
                ok, v - lo, jnp.full((16,), NR, jnp.int32))


def _sc_scatter_msg_body(dst3d_hbm, msg_hbm, zeros_hbm, agg_hbm,
                         didx_v, didx_r, dat_v, acc_sp):
    cid = lax.axis_index("c")
    sid = lax.axis_index("s")
    pltpu.sync_copy(dst3d_hbm.at[sid], didx_v)
    lo = cid * NR
    _rewrite_range_idx(didx_v, didx_r, lo)
    pltpu.sync_copy(zeros_hbm, acc_sp.at[pl.ds(sid * RPT, RPT)])
    plsc.subcore_barrier()
    for j in range(EPT // IB):
        pltpu.sync_copy(msg_hbm.at[pl.ds(sid * EPT + j * IB, IB)], dat_v)
        pltpu.sync_copy(dat_v, acc_sp.at[didx_r.at[j]], add=True)
    plsc.subcore_barrier()
    pltpu.sync_copy(acc_sp.at[pl.ds(sid * RPT, RPT)],
                    agg_hbm.at[pl.ds(lo + sid * RPT, RPT)])


def _make_sc_gcn(mesh):
    return functools.partial(
        pl.kernel,
        out_type=jax.ShapeDtypeStruct((N, 128), jnp.bfloat16),
        mesh=mesh,
        scratch_types=(
            pltpu.VMEM((EPT // IB, IB), jnp.int32),
            pltpu.VMEM((EPT // IB, IB), jnp.int32),
            pltpu.VMEM((EPT // IB, IB), jnp.int32),
            pltpu.VMEM((IB, 128), jnp.bfloat16),
            pltpu.VMEM_SHARED((NR + 8, 128), jnp.bfloat16),
            pltpu.SemaphoreType.DMA,
        ),
        compiler_params=_SC_PARAMS,
    )(_sc_gcn_body)


def _sc_gcn_body(src3d_hbm, dst3d_hbm, ns_hbm, zeros_hbm, s_hbm,
                 sidx_v, didx_v, didx_r, dat_v, acc_sp, sem):
    cid = lax.axis_index("c")
    sid = lax.axis_index("s")
    pltpu.sync_copy(src3d_hbm.at[sid], sidx_v)
    pltpu.sync_copy(dst3d_hbm.at[sid], didx_v)
    lo = cid * NR
    _rewrite_range_idx(didx_v, didx_r, lo)
    pltpu.sync_copy(zeros_hbm, acc_sp.at[pl.ds(sid * RPT, RPT)])
    plsc.subcore_barrier()
    for j in range(EPT // IB):
        pltpu.async_copy(ns_hbm.at[sidx_v.at[j]], dat_v, sem).wait()
        pltpu.sync_copy(dat_v, acc_sp.at[didx_r.at[j]], add=True)
    plsc.subcore_barrier()
    pltpu.sync_copy(acc_sp.at[pl.ds(sid * RPT, RPT)],
                    s_hbm.at[pl.ds(lo + sid * RPT, RPT)])


EB = 2048


def _t1_body(ea_ref, xs_ref, w1_ref, b1_ref, w2_ref, b2_ref, o_ref):
    h1 = jnp.maximum(
        jnp.dot(ea_ref[...], w1_ref[...],
                preferred_element_type=jnp.float32) + b1_ref[...], 0.0)
    xs = xs_ref[...]
    msg = jnp.zeros((EB, 128), jnp.float32)
    for i in range(7):
        h2i = jnp.maximum(
            jnp.dot(h1, w2_ref[:, 128 * i:128 * (i + 1)],
                    preferred_element_type=jnp.float32)
            + b2_ref[:, 128 * i:128 * (i + 1)], 0.0)
        msg = msg + xs[:, i:i + 1] * h2i
    o_ref[...] = msg.astype(jnp.bfloat16)


def _t1(ea, xsrc, W1, b1, W2, b2):
    return pl.pallas_call(
        _t1_body,
        grid=(E // EB,),
        in_specs=[
            pl.BlockSpec((EB, 19), lambda e: (e, 0)),
            pl.BlockSpec((EB, 128), lambda e: (e, 0)),
            pl.BlockSpec((19, 128), lambda e: (0, 0)),
            pl.BlockSpec((1, 128), lambda e: (0, 0)),
            pl.BlockSpec((128, 896), lambda e: (0, 0)),
            pl.BlockSpec((1, 896), lambda e: (0, 0)),
        ],
        out_specs=pl.BlockSpec((EB, 128), lambda e: (e, 0)),
        out_shape=jax.ShapeDtypeStruct((E, 128), jnp.bfloat16),
    )(ea, xsrc, W1, b1, W2, b2)


NB = 4096


def _t2_body(xp_ref, agg_ref, deg_ref, rw_ref, rb_ref, ns_ref):
    node = (jnp.dot(xp_ref[...], rw_ref[...],
                    preferred_element_type=jnp.float32)
            + rb_ref[...] + agg_ref[...].astype(jnp.float32))
    dinv = lax.rsqrt(deg_ref[:, 0:1] + 1.0)
    ns_ref[...] = (dinv * node).astype(jnp.bfloat16)


def _t2(xpad, agg, deg, rootpad, root_b):
    f32 = jnp.float32
    return pl.pallas_call(
        _t2_body,
        grid=(N // NB,),
        in_specs=[
            pl.BlockSpec((NB, 128), lambda n: (n, 0)),
            pl.BlockSpec((NB, 128), lambda n: (n, 0)),
            pl.BlockSpec((NB, 128), lambda n: (n, 0)),
            pl.BlockSpec((128, 128), lambda n: (0, 0)),
            pl.BlockSpec((1, 128), lambda n: (0, 0)),
        ],
        out_specs=pl.BlockSpec((NB, 128), lambda n: (n, 0)),
        out_shape=jax.ShapeDtypeStruct((N, 128), jnp.bfloat16),
    )(xpad, agg, deg, rootpad, root_b)


def _t3_body(s_ref, ns_ref, deg_ref, gw_ref, gb_ref, out_ref):
    dinv = lax.rsqrt(deg_ref[:, 0:1] + 1.0)
    pre = dinv * (s_ref[...].astype(jnp.float32)
                  + ns_ref[...].astype(jnp.float32))
    out_ref[...] = jnp.maximum(
        jnp.dot(pre, gw_ref[...], preferred_element_type=jnp.float32)
        + gb_ref[...], 0.0)


def _t3(S, ns, deg, gcn_w, gcn_b):
    return pl.pallas_call(
        _t3_body,
        grid=(N // NB,),
        in_specs=[
            pl.BlockSpec((NB, 128), lambda n: (n, 0)),
            pl.BlockSpec((NB, 128), lambda n: (n, 0)),
            pl.BlockSpec((NB, 128), lambda n: (n, 0)),
            pl.BlockSpec((128, 256), lambda n: (0, 0)),
            pl.BlockSpec((1, 256), lambda n: (0, 0)),
        ],
        out_specs=pl.BlockSpec((NB, 256), lambda n: (n, 0)),
        out_shape=jax.ShapeDtypeStruct((N, 256), jnp.float32),
    )(S, ns, deg, gcn_w, gcn_b)


GB = 1024


def _t4_body(g_ref, w1, b1, w2, b2, w3, b3, pw, pb, out_ref):
    t = jnp.maximum(
        jnp.dot(g_ref[...], w1[...], preferred_element_type=jnp.float32)
        + b1[...], 0.0)
    t = jnp.maximum(
        jnp.dot(t, w2[...], preferred_element_type=jnp.float32) + b2[...],
        0.0)
    t = jnp.maximum(
        jnp.dot(t, w3[...], preferred_element_type=jnp.float32) + b3[...],
        0.0)
    out_ref[...] = (jnp.dot(t, pw[...], preferred_element_type=jnp.float32)
                    + pb[...])


def _t4(g, f1w, f1b, f2w, f2b, f3w, f3b, pw, pb):
    NG = N // 6
    return pl.pallas_call(
        _t4_body,
        grid=(NG // GB,),
        in_specs=[
            pl.BlockSpec((GB, 1536), lambda n: (n, 0)),
            pl.BlockSpec((1536, 512), lambda n: (0, 0)),
            pl.BlockSpec((1, 512), lambda n: (0, 0)),
            pl.BlockSpec((512, 256), lambda n: (0, 0)),
            pl.BlockSpec((1, 256), lambda n: (0, 0)),
            pl.BlockSpec((256, 128), lambda n: (0, 0)),
            pl.BlockSpec((1, 128), lambda n: (0, 0)),
            pl.BlockSpec((128, 7), lambda n: (0, 0)),
            pl.BlockSpec((1, 7), lambda n: (0, 0)),
        ],
        out_specs=pl.BlockSpec((GB, 7), lambda n: (n, 0)),
        out_shape=jax.ShapeDtypeStruct((NG, 7), jnp.float32),
    )(g, f1w, f1b, f2w, f2b, f3w, f3b, pw, pb)


def kernel(x, edge_index, edge_attr, num_graphs, W1, b1, W2, b2, root,
           root_b, gcn_w, gcn_b, f1w, f1b, f2w, f2b, f3w, f3b, pw, pb):
    f32 = jnp.float32
    src = edge_index[0].astype(jnp.int32)
    dst = edge_index[1].astype(jnp.int32)
    src3d_w = src.reshape(NW, EPW // IB, IB)
    src3d = src.reshape(NS, EPT // IB, IB)
    dst3d = dst.reshape(NS, EPT // IB, IB)
    xpad = jnp.pad(x, ((0, 0), (0, 121)))
    rootpad = jnp.pad(root, ((0, 121), (0, 0)))
    ones8 = jnp.ones((IB, 8), f32)
    zeros8 = jnp.zeros((NPT, 8), f32)
    zerosr = jnp.zeros((RPT, 128), jnp.bfloat16)

    sc_gather_deg, sc_scatter_msg, sc_gcn = _sc_kernels()
    xsrc, deg = sc_gather_deg(xpad, src3d_w, dst3d, ones8, zeros8)
    msg = _t1(edge_attr, xsrc, W1, b1.reshape(1, 128), W2,
              b2.reshape(1, 896))
    agg = sc_scatter_msg(dst3d, msg, zerosr)
    ns = _t2(xpad, agg, deg, rootpad, root_b.reshape(1, 128))
    S = sc_gcn(src3d, dst3d, ns, zerosr)
    h2g = _t3(S, ns, deg, gcn_w, gcn_b.reshape(1, 256))
    g = h2g.reshape(N // 6, 1536)
    return _t4(g, f1w, f1b.reshape(1, 512), f2w, f2b.reshape(1, 256),
               f3w, f3b.reshape(1, 128), pw, pb.reshape(1, 7))


# speedup vs baseline: 1.1185x; 1.1185x over previous
import functools

import jax
import jax.numpy as jnp
from jax import lax
from jax.experimental import pallas as pl
from jax.experimental.pallas import tpu as pltpu
from jax.experimental.pallas import tpu_sc as plsc

N = 8192 * 6
E = 8192 * 5
NC, NS = 2, 16
NW = NC * NS
EPW = E // NW
EPT = E // NS
NPT = N // NS
IB = 128
NRANGE = 4
NR = N // NRANGE
RPT = NR // NS

_SC_PARAMS = pltpu.CompilerParams(use_tc_tiling_on_sc=False)


@functools.cache
def _sc_kernels():
    mesh = plsc.VectorSubcoreMesh(
        core_axis_name="c", subcore_axis_name="s",
        num_cores=NC, num_subcores=NS)
    g0 = _make_sc_gather_deg(mesh)
    s2 = _make_sc_scatter_msg(mesh)
    s3 = _make_sc_gcn(mesh)
    return g0, s2, s3


def _make_sc_gather_deg(mesh):
    return functools.partial(
        pl.kernel,
        out_type=(jax.ShapeDtypeStruct((E, 128), jnp.float32),
                  jax.ShapeDtypeStruct((N, 128), jnp.float32)),
        mesh=mesh,
        scratch_types=(
            pltpu.VMEM((EPW // IB, IB), jnp.int32),
            pltpu.VMEM((EPT // IB, IB), jnp.int32),
            pltpu.VMEM((IB, 128), jnp.float32),
            pltpu.VMEM((IB, 8), jnp.float32),
            pltpu.VMEM_SHARED((N, 8), jnp.float32),
            pltpu.SemaphoreType.DMA,
        ),
        compiler_params=_SC_PARAMS,
    )(_sc_gather_deg_body)


def _sc_gather_deg_body(xpad_hbm, src3d_hbm, dst3d_hbm, ones_hbm, zeros8_hbm,
                        xsrc_hbm, deg_hbm,
                        sidx_v, didx_v, rows_v, ones_v, deg_sp, sem):
    cid = lax.axis_index("c")
    sid = lax.axis_index("s")
    wid = sid * NC + cid
    pltpu.sync_copy(src3d_hbm.at[wid], sidx_v)
    for j in range(EPW // IB):
        pltpu.async_copy(xpad_hbm.at[sidx_v.at[j]], rows_v, sem).wait()
        pltpu.sync_copy(rows_v, xsrc_hbm.at[pl.ds(wid * EPW + j * IB, IB)])

    @pl.when(cid == 0)
    def _deg():
        pltpu.sync_copy(zeros8_hbm, deg_sp.at[pl.ds(sid * NPT, NPT)])
        pltpu.sync_copy(dst3d_hbm.at[sid], didx_v)
        pltpu.sync_copy(ones_hbm, ones_v)
        plsc.subcore_barrier()
        for j in range(EPT // IB):
            pltpu.sync_copy(ones_v, deg_sp.at[didx_v.at[j]], add=True)
        plsc.subcore_barrier()
        pltpu.sync_copy(deg_sp.at[pl.ds(sid * NPT, NPT)],
                        deg_hbm.at[pl.ds(sid * NPT, NPT), pl.ds(0, 8)])


def _make_sc_scatter_msg(mesh):
    return functools.partial(
        pl.kernel,
        out_type=jax.ShapeDtypeStruct((N, 128), jnp.float32),
        mesh=mesh,
        scratch_types=(
            pltpu.VMEM((EPT // IB, IB), jnp.int32),
            pltpu.VMEM((EPT // IB, IB), jnp.int32),
            pltpu.VMEM((IB, 128), jnp.float32),
            pltpu.VMEM_SHARED((NR + 8, 128), jnp.float32),
        ),
        compiler_params=_SC_PARAMS,
    )(_sc_scatter_msg_body)


def _rewrite_range_idx(didx_v, didx_r, lo):
    hi = lo + NR
    for r in range(EPT // IB):
        for k in range(IB // 16):
            v = didx_v[r, pl.ds(16 * k, 16)]
            ok = (v >= lo) & (v < hi)
            didx_r[r, pl.ds(16 * k, 16)] = jnp.where(
                ok, v - lo, jnp.full((16,), NR, jnp.int32))


def _sc_scatter_msg_body(dst3d_hbm, msg_hbm, zeros_hbm, agg_hbm,
                         didx_v, didx_r, dat_v, acc_sp):
    cid = lax.axis_index("c")
    sid = lax.axis_index("s")
    pltpu.sync_copy(dst3d_hbm.at[sid], didx_v)
    for p in range(2):
        lo = (2 * cid + p) * NR
        _rewrite_range_idx(didx_v, didx_r, lo)
        pltpu.sync_copy(zeros_hbm, acc_sp.at[pl.ds(sid * RPT, RPT)])
        plsc.subcore_barrier()
        for j in range(EPT // IB):
            pltpu.sync_copy(msg_hbm.at[pl.ds(sid * EPT + j * IB, IB)], dat_v)
            pltpu.sync_copy(dat_v, acc_sp.at[didx_r.at[j]], add=True)
        plsc.subcore_barrier()
        pltpu.sync_copy(acc_sp.at[pl.ds(sid * RPT, RPT)],
                        agg_hbm.at[pl.ds(lo + sid * RPT, RPT)])
        plsc.subcore_barrier()


def _make_sc_gcn(mesh):
    return functools.partial(
        pl.kernel,
        out_type=jax.ShapeDtypeStruct((N, 128), jnp.float32),
        mesh=mesh,
        scratch_types=(
            pltpu.VMEM((EPT // IB, IB), jnp.int32),
            pltpu.VMEM((EPT // IB, IB), jnp.int32),
            pltpu.VMEM((EPT // IB, IB), jnp.int32),
            pltpu.VMEM((IB, 128), jnp.float32),
            pltpu.VMEM_SHARED((NR + 8, 128), jnp.float32),
            pltpu.SemaphoreType.DMA,
        ),
        compiler_params=_SC_PARAMS,
    )(_sc_gcn_body)


def _sc_gcn_body(src3d_hbm, dst3d_hbm, ns_hbm, zeros_hbm, s_hbm,
                 sidx_v, didx_v, didx_r, dat_v, acc_sp, sem):
    cid = lax.axis_index("c")
    sid = lax.axis_index("s")
    pltpu.sync_copy(src3d_hbm.at[sid], sidx_v)
    pltpu.sync_copy(dst3d_hbm.at[sid], didx_v)
    for p in range(2):
        lo = (2 * cid + p) * NR
        _rewrite_range_idx(didx_v, didx_r, lo)
        pltpu.sync_copy(zeros_hbm, acc_sp.at[pl.ds(sid * RPT, RPT)])
        plsc.subcore_barrier()
        for j in range(EPT // IB):
            pltpu.async_copy(ns_hbm.at[sidx_v.at[j]], dat_v, sem).wait()
            pltpu.sync_copy(dat_v, acc_sp.at[didx_r.at[j]], add=True)
        plsc.subcore_barrier()
        pltpu.sync_copy(acc_sp.at[pl.ds(sid * RPT, RPT)],
                        s_hbm.at[pl.ds(lo + sid * RPT, RPT)])
        plsc.subcore_barrier()


EB = 2048


def _t1_body(ea_ref, xs_ref, w1_ref, b1_ref, w2_ref, b2_ref, o_ref):
    bf16 = jnp.bfloat16
    h1 = jnp.maximum(
        jnp.dot(ea_ref[...], w1_ref[...],
                preferred_element_type=jnp.float32) + b1_ref[...], 0.0)
    h1b = h1.astype(bf16)
    w2b = w2_ref[...].astype(bf16)
    xs = xs_ref[...]
    msg = jnp.zeros((EB, 128), jnp.float32)
    for i in range(7):
        h2i = jnp.maximum(
            jnp.dot(h1b, w2b[:, 128 * i:128 * (i + 1)],
                    preferred_element_type=jnp.float32)
            + b2_ref[:, 128 * i:128 * (i + 1)], 0.0)
        msg = msg + xs[:, i:i + 1] * h2i
    o_ref[...] = msg


def _t1(ea, xsrc, W1, b1, W2, b2):
    return pl.pallas_call(
        _t1_body,
        grid=(E // EB,),
        in_specs=[
            pl.BlockSpec((EB, 19), lambda e: (e, 0)),
            pl.BlockSpec((EB, 128), lambda e: (e, 0)),
            pl.BlockSpec((19, 128), lambda e: (0, 0)),
            pl.BlockSpec((1, 128), lambda e: (0, 0)),
            pl.BlockSpec((128, 896), lambda e: (0, 0)),
            pl.BlockSpec((1, 896), lambda e: (0, 0)),
        ],
        out_specs=pl.BlockSpec((EB, 128), lambda e: (e, 0)),
        out_shape=jax.ShapeDtypeStruct((E, 128), jnp.float32),
    )(ea, xsrc, W1, b1, W2, b2)


NB = 4096


def _t2_body(xp_ref, agg_ref, deg_ref, rw_ref, rb_ref, ns_ref):
    node = (jnp.dot(xp_ref[...], rw_ref[...],
                    preferred_element_type=jnp.float32)
            + rb_ref[...] + agg_ref[...])
    dinv = lax.rsqrt(deg_ref[:, 0:1] + 1.0)
    ns_ref[...] = dinv * node


def _t2(xpad, agg, deg, rootpad, root_b):
    f32 = jnp.float32
    return pl.pallas_call(
        _t2_body,
        grid=(N // NB,),
        in_specs=[
            pl.BlockSpec((NB, 128), lambda n: (n, 0)),
            pl.BlockSpec((NB, 128), lambda n: (n, 0)),
            pl.BlockSpec((NB, 128), lambda n: (n, 0)),
            pl.BlockSpec((128, 128), lambda n: (0, 0)),
            pl.BlockSpec((1, 128), lambda n: (0, 0)),
        ],
        out_specs=pl.BlockSpec((NB, 128), lambda n: (n, 0)),
        out_shape=jax.ShapeDtypeStruct((N, 128), f32),
    )(xpad, agg, deg, rootpad, root_b)


def _t3_body(s_ref, ns_ref, deg_ref, gw_ref, gb_ref, out_ref):
    dinv = lax.rsqrt(deg_ref[:, 0:1] + 1.0)
    pre = dinv * (s_ref[...] + ns_ref[...])
    out_ref[...] = jnp.maximum(
        jnp.dot(pre, gw_ref[...], preferred_element_type=jnp.float32)
        + gb_ref[...], 0.0).astype(jnp.bfloat16)


def _t3(S, ns, deg, gcn_w, gcn_b):
    return pl.pallas_call(
        _t3_body,
        grid=(N // NB,),
        in_specs=[
            pl.BlockSpec((NB, 128), lambda n: (n, 0)),
            pl.BlockSpec((NB, 128), lambda n: (n, 0)),
            pl.BlockSpec((NB, 128), lambda n: (n, 0)),
            pl.BlockSpec((128, 256), lambda n: (0, 0)),
            pl.BlockSpec((1, 256), lambda n: (0, 0)),
        ],
        out_specs=pl.BlockSpec((NB, 256), lambda n: (n, 0)),
        out_shape=jax.ShapeDtypeStruct((N, 256), jnp.bfloat16),
    )(S, ns, deg, gcn_w, gcn_b)


GB = 1024


def _t4_body(g_ref, w1, b1, w2, b2, w3, b3, pw, pb, out_ref):
    bf16 = jnp.bfloat16
    t = jnp.maximum(
        jnp.dot(g_ref[...], w1[...].astype(bf16),
                preferred_element_type=jnp.float32) + b1[...], 0.0)
    t = jnp.maximum(
        jnp.dot(t.astype(bf16), w2[...].astype(bf16),
                preferred_element_type=jnp.float32) + b2[...], 0.0)
    t = jnp.maximum(
        jnp.dot(t.astype(bf16), w3[...].astype(bf16),
                preferred_element_type=jnp.float32) + b3[...], 0.0)
    out_ref[...] = (jnp.dot(t, pw[...], preferred_element_type=jnp.float32)
                    + pb[...])


def _t4(g, f1w, f1b, f2w, f2b, f3w, f3b, pw, pb):
    NG = N // 6
    return pl.pallas_call(
        _t4_body,
        grid=(NG // GB,),
        in_specs=[
            pl.BlockSpec((GB, 1536), lambda n: (n, 0)),
            pl.BlockSpec((1536, 512), lambda n: (0, 0)),
            pl.BlockSpec((1, 512), lambda n: (0, 0)),
            pl.BlockSpec((512, 256), lambda n: (0, 0)),
            pl.BlockSpec((1, 256), lambda n: (0, 0)),
            pl.BlockSpec((256, 128), lambda n: (0, 0)),
            pl.BlockSpec((1, 128), lambda n: (0, 0)),
            pl.BlockSpec((128, 7), lambda n: (0, 0)),
            pl.BlockSpec((1, 7), lambda n: (0, 0)),
        ],
        out_specs=pl.BlockSpec((GB, 7), lambda n: (n, 0)),
        out_shape=jax.ShapeDtypeStruct((NG, 7), jnp.float32),
    )(g, f1w, f1b, f2w, f2b, f3w, f3b, pw, pb)


def kernel(x, edge_index, edge_attr, num_graphs, W1, b1, W2, b2, root,
           root_b, gcn_w, gcn_b, f1w, f1b, f2w, f2b, f3w, f3b, pw, pb):
    f32 = jnp.float32
    src = edge_index[0].astype(jnp.int32)
    dst = edge_index[1].astype(jnp.int32)
    src3d_w = src.reshape(NW, EPW // IB, IB)
    src3d = src.reshape(NS, EPT // IB, IB)
    dst3d = dst.reshape(NS, EPT // IB, IB)
    xpad = jnp.pad(x, ((0, 0), (0, 121)))
    rootpad = jnp.pad(root, ((0, 121), (0, 0)))
    ones8 = jnp.ones((IB, 8), f32)
    zeros8 = jnp.zeros((NPT, 8), f32)
    zerosr = jnp.zeros((RPT, 128), f32)

    sc_gather_deg, sc_scatter_msg, sc_gcn = _sc_kernels()
    xsrc, deg = sc_gather_deg(xpad, src3d_w, dst3d, ones8, zeros8)
    msg = _t1(edge_attr, xsrc, W1, b1.reshape(1, 128), W2,
              b2.reshape(1, 896))
    agg = sc_scatter_msg(dst3d, msg, zerosr)
    ns = _t2(xpad, agg, deg, rootpad, root_b.reshape(1, 128))
    S = sc_gcn(src3d, dst3d, ns, zerosr)
    h2g = _t3(S, ns, deg, gcn_w, gcn_b.reshape(1, 256))
    g = h2g.reshape(N // 6, 1536)
    return _t4(g, f1w, f1b.reshape(1, 512), f2w, f2b.reshape(1, 256),
               f3w, f3b.reshape(1, 128), pw, pb.reshape(1, 7))


# speedup vs baseline: 1.1538x; 1.0316x over previous
import functools

import jax
import jax.numpy as jnp
from jax import lax
from jax.experimental import pallas as pl
from jax.experimental.pallas import tpu as pltpu
from jax.experimental.pallas import tpu_sc as plsc

N = 8192 * 6
E = 8192 * 5
NC, NS = 2, 16
NW = NC * NS
EPW = E // NW
EPT = E // NS
NPT = N // NS
IB = 128
IBS = 64
NRANGE = 4
NR = N // NRANGE
RPT = NR // NS

_SC_PARAMS = pltpu.CompilerParams(use_tc_tiling_on_sc=False)


@functools.cache
def _sc_kernels():
    mesh = plsc.VectorSubcoreMesh(
        core_axis_name="c", subcore_axis_name="s",
        num_cores=NC, num_subcores=NS)
    g0 = _make_sc_gather_deg(mesh)
    s2 = _make_sc_scatter_msg(mesh)
    s3 = _make_sc_gcn(mesh)
    return g0, s2, s3


def _make_sc_gather_deg(mesh):
    return functools.partial(
        pl.kernel,
        out_type=(jax.ShapeDtypeStruct((E, 128), jnp.float32),
                  jax.ShapeDtypeStruct((N, 128), jnp.float32)),
        mesh=mesh,
        scratch_types=(
            pltpu.VMEM((EPW // IB, IB), jnp.int32),
            pltpu.VMEM((EPT // IBS, IBS), jnp.int32),
            pltpu.VMEM((IB, 128), jnp.float32),
            pltpu.VMEM((IB, 128), jnp.float32),
            pltpu.VMEM((IBS, 8), jnp.float32),
            pltpu.VMEM_SHARED((N, 8), jnp.float32),
            pltpu.SemaphoreType.DMA,
            pltpu.SemaphoreType.DMA,
            pltpu.SemaphoreType.DMA,
            pltpu.SemaphoreType.DMA,
        ),
        compiler_params=_SC_PARAMS,
    )(_sc_gather_deg_body)


def _sc_gather_deg_body(xpad_hbm, src3d_hbm, dst3d_hbm, ones_hbm, zeros8_hbm,
                        xsrc_hbm, deg_hbm,
                        sidx_v, didx_v, rows_v0, rows_v1, ones_v, deg_sp,
                        gsem0, gsem1, wsem0, wsem1):
    cid = lax.axis_index("c")
    sid = lax.axis_index("s")
    wid = sid * NC + cid
    rows_v = (rows_v0, rows_v1)
    gsem = (gsem0, gsem1)
    wsem = (wsem0, wsem1)
    nj = EPW // IB
    pltpu.sync_copy(src3d_hbm.at[wid], sidx_v)
    gd = [None, None]
    wd = [None, None]
    for j in range(nj):
        b = j % 2
        if wd[b] is not None:
            wd[b].wait()
        gd[b] = pltpu.async_copy(xpad_hbm.at[sidx_v.at[j]], rows_v[b],
                                 gsem[b])
        if j > 0:
            pb = (j - 1) % 2
            gd[pb].wait()
            wd[pb] = pltpu.async_copy(
                rows_v[pb],
                xsrc_hbm.at[pl.ds(wid * EPW + (j - 1) * IB, IB)], wsem[pb])
    lb = (nj - 1) % 2
    gd[lb].wait()
    wd[lb] = pltpu.async_copy(
        rows_v[lb], xsrc_hbm.at[pl.ds(wid * EPW + (nj - 1) * IB, IB)],
        wsem[lb])
    wd[0].wait()
    wd[1].wait()

    @pl.when(cid == 0)
    def _deg():
        pltpu.sync_copy(zeros8_hbm, deg_sp.at[pl.ds(sid * NPT, NPT)])
        pltpu.sync_copy(dst3d_hbm.at[sid], didx_v)
        pltpu.sync_copy(ones_hbm, ones_v)
        plsc.subcore_barrier()
        for j in range(EPT // IBS):
            pltpu.sync_copy(ones_v, deg_sp.at[didx_v.at[j]], add=True)
        plsc.subcore_barrier()
        pltpu.sync_copy(deg_sp.at[pl.ds(sid * NPT, NPT)],
                        deg_hbm.at[pl.ds(sid * NPT, NPT), pl.ds(0, 8)])


def _make_sc_scatter_msg(mesh):
    return functools.partial(
        pl.kernel,
        out_type=jax.ShapeDtypeStruct((N, 128), jnp.float32),
        mesh=mesh,
        scratch_types=(
            pltpu.VMEM((EPT // IBS, IBS), jnp.int32),
            pltpu.VMEM((EPT // IBS, IBS), jnp.int32),
            pltpu.VMEM((IBS, 128), jnp.float32),
            pltpu.VMEM((IBS, 128), jnp.float32),
            pltpu.VMEM_SHARED((NR + 8, 128), jnp.float32),
            pltpu.SemaphoreType.DMA,
            pltpu.SemaphoreType.DMA,
        ),
        compiler_params=_SC_PARAMS,
    )(_sc_scatter_msg_body)


def _rewrite_range_idx(didx_v, didx_r, lo):
    hi = lo + NR
    for r in range(EPT // IBS):
        for k in range(IBS // 16):
            v = didx_v[r, pl.ds(16 * k, 16)]
            ok = (v >= lo) & (v < hi)
            didx_r[r, pl.ds(16 * k, 16)] = jnp.where(
                ok, v - lo, jnp.full((16,), NR, jnp.int32))


def _sc_scatter_msg_body(dst3d_hbm, msg_hbm, zeros_hbm, agg_hbm,
                         didx_v, didx_r, dat_v0, dat_v1, acc_sp,
                         lsem0, lsem1):
    cid = lax.axis_index("c")
    sid = lax.axis_index("s")
    dat_v = (dat_v0, dat_v1)
    lsem = (lsem0, lsem1)
    nj = EPT // IBS
    pltpu.sync_copy(dst3d_hbm.at[sid], didx_v)
    for p in range(2):
        lo = (2 * cid + p) * NR
        _rewrite_range_idx(didx_v, didx_r, lo)
        pltpu.sync_copy(zeros_hbm, acc_sp.at[pl.ds(sid * RPT, RPT)])
        plsc.subcore_barrier()
        ld = [None, None]
        ld[0] = pltpu.async_copy(msg_hbm.at[pl.ds(sid * EPT, IBS)], dat_v[0],
                                 lsem[0])
        for j in range(nj):
            b = j % 2
            if j + 1 < nj:
                nb = (j + 1) % 2
                ld[nb] = pltpu.async_copy(
                    msg_hbm.at[pl.ds(sid * EPT + (j + 1) * IBS, IBS)],
                    dat_v[nb], lsem[nb])
            ld[b].wait()
            pltpu.sync_copy(dat_v[b], acc_sp.at[didx_r.at[j]], add=True)
        plsc.subcore_barrier()
        pltpu.sync_copy(acc_sp.at[pl.ds(sid * RPT, RPT)],
                        agg_hbm.at[pl.ds(lo + sid * RPT, RPT)])
        plsc.subcore_barrier()


def _make_sc_gcn(mesh):
    return functools.partial(
        pl.kernel,
        out_type=jax.ShapeDtypeStruct((N, 128), jnp.float32),
        mesh=mesh,
        scratch_types=(
            pltpu.VMEM((EPT // IBS, IBS), jnp.int32),
            pltpu.VMEM((EPT // IBS, IBS), jnp.int32),
            pltpu.VMEM((EPT // IBS, IBS), jnp.int32),
            pltpu.VMEM((IBS, 128), jnp.float32),
            pltpu.VMEM((IBS, 128), jnp.float32),
            pltpu.VMEM_SHARED((NR + 8, 128), jnp.float32),
            pltpu.SemaphoreType.DMA,
            pltpu.SemaphoreType.DMA,
        ),
        compiler_params=_SC_PARAMS,
    )(_sc_gcn_body)


def _sc_gcn_body(src3d_hbm, dst3d_hbm, ns_hbm, zeros_hbm, s_hbm,
                 sidx_v, didx_v, didx_r, dat_v0, dat_v1, acc_sp,
                 lsem0, lsem1):
    cid = lax.axis_index("c")
    sid = lax.axis_index("s")
    dat_v = (dat_v0, dat_v1)
    lsem = (lsem0, lsem1)
    nj = EPT // IBS
    pltpu.sync_copy(src3d_hbm.at[sid], sidx_v)
    pltpu.sync_copy(dst3d_hbm.at[sid], didx_v)
    for p in range(2):
        lo = (2 * cid + p) * NR
        _rewrite_range_idx(didx_v, didx_r, lo)
        pltpu.sync_copy(zeros_hbm, acc_sp.at[pl.ds(sid * RPT, RPT)])
        plsc.subcore_barrier()
        ld = [None, None]
        ld[0] = pltpu.async_copy(ns_hbm.at[sidx_v.at[0]], dat_v[0], lsem[0])
        for j in range(nj):
            b = j % 2
            if j + 1 < nj:
                nb = (j + 1) % 2
                ld[nb] = pltpu.async_copy(ns_hbm.at[sidx_v.at[j + 1]],
                                          dat_v[nb], lsem[nb])
            ld[b].wait()
            pltpu.sync_copy(dat_v[b], acc_sp.at[didx_r.at[j]], add=True)
        plsc.subcore_barrier()
        pltpu.sync_copy(acc_sp.at[pl.ds(sid * RPT, RPT)],
                        s_hbm.at[pl.ds(lo + sid * RPT, RPT)])
        plsc.subcore_barrier()


EB = 2048


def _t1_body(ea_ref, xs_ref, w1_ref, b1_ref, w2_ref, b2_ref, o_ref):
    bf16 = jnp.bfloat16
    h1 = jnp.maximum(
        jnp.dot(ea_ref[...], w1_ref[...],
                preferred_element_type=jnp.float32) + b1_ref[...], 0.0)
    h1b = h1.astype(bf16)
    w2b = w2_ref[...].astype(bf16)
    xs = xs_ref[...]
    msg = jnp.zeros((EB, 128), jnp.float32)
    for i in range(7):
        h2i = jnp.maximum(
            jnp.dot(h1b, w2b[:, 128 * i:128 * (i + 1)],
                    preferred_element_type=jnp.float32)
            + b2_ref[:, 128 * i:128 * (i + 1)], 0.0)
        msg = msg + xs[:, i:i + 1] * h2i
    o_ref[...] = msg


def _t1(ea, xsrc, W1, b1, W2, b2):
    return pl.pallas_call(
        _t1_body,
        grid=(E // EB,),
        in_specs=[
            pl.BlockSpec((EB, 19), lambda e: (e, 0)),
            pl.BlockSpec((EB, 128), lambda e: (e, 0)),
            pl.BlockSpec((19, 128), lambda e: (0, 0)),
            pl.BlockSpec((1, 128), lambda e: (0, 0)),
            pl.BlockSpec((128, 896), lambda e: (0, 0)),
            pl.BlockSpec((1, 896), lambda e: (0, 0)),
        ],
        out_specs=pl.BlockSpec((EB, 128), lambda e: (e, 0)),
        out_shape=jax.ShapeDtypeStruct((E, 128), jnp.float32),
    )(ea, xsrc, W1, b1, W2, b2)


NB = 4096


def _t2_body(xp_ref, agg_ref, deg_ref, rw_ref, rb_ref, ns_ref):
    node = (jnp.dot(xp_ref[...], rw_ref[...],
                    preferred_element_type=jnp.float32)
            + rb_ref[...] + agg_ref[...])
    dinv = lax.rsqrt(deg_ref[:, 0:1] + 1.0)
    ns_ref[...] = dinv * node


def _t2(xpad, agg, deg, rootpad, root_b):
    f32 = jnp.float32
    return pl.pallas_call(
        _t2_body,
        grid=(N // NB,),
        in_specs=[
            pl.BlockSpec((NB, 128), lambda n: (n, 0)),
            pl.BlockSpec((NB, 128), lambda n: (n, 0)),
            pl.BlockSpec((NB, 128), lambda n: (n, 0)),
            pl.BlockSpec((128, 128), lambda n: (0, 0)),
            pl.BlockSpec((1, 128), lambda n: (0, 0)),
        ],
        out_specs=pl.BlockSpec((NB, 128), lambda n: (n, 0)),
        out_shape=jax.ShapeDtypeStruct((N, 128), f32),
    )(xpad, agg, deg, rootpad, root_b)


def _t3_body(s_ref, ns_ref, deg_ref, gw_ref, gb_ref, out_ref):
    dinv = lax.rsqrt(deg_ref[:, 0:1] + 1.0)
    pre = dinv * (s_ref[...] + ns_ref[...])
    out_ref[...] = jnp.maximum(
        jnp.dot(pre, gw_ref[...], preferred_element_type=jnp.float32)
        + gb_ref[...], 0.0).astype(jnp.bfloat16)


def _t3(S, ns, deg, gcn_w, gcn_b):
    return pl.pallas_call(
        _t3_body,
        grid=(N // NB,),
        in_specs=[
            pl.BlockSpec((NB, 128), lambda n: (n, 0)),
            pl.BlockSpec((NB, 128), lambda n: (n, 0)),
            pl.BlockSpec((NB, 128), lambda n: (n, 0)),
            pl.BlockSpec((128, 256), lambda n: (0, 0)),
            pl.BlockSpec((1, 256), lambda n: (0, 0)),
        ],
        out_specs=pl.BlockSpec((NB, 256), lambda n: (n, 0)),
        out_shape=jax.ShapeDtypeStruct((N, 256), jnp.bfloat16),
    )(S, ns, deg, gcn_w, gcn_b)


GB = 1024


def _t4_body(g_ref, w1, b1, w2, b2, w3, b3, pw, pb, out_ref):
    bf16 = jnp.bfloat16
    t = jnp.maximum(
        jnp.dot(g_ref[...], w1[...].astype(bf16),
                preferred_element_type=jnp.float32) + b1[...], 0.0)
    t = jnp.maximum(
        jnp.dot(t.astype(bf16), w2[...].astype(bf16),
                preferred_element_type=jnp.float32) + b2[...], 0.0)
    t = jnp.maximum(
        jnp.dot(t.astype(bf16), w3[...].astype(bf16),
                preferred_element_type=jnp.float32) + b3[...], 0.0)
    out_ref[...] = (jnp.dot(t, pw[...], preferred_element_type=jnp.float32)
                    + pb[...])


def _t4(g, f1w, f1b, f2w, f2b, f3w, f3b, pw, pb):
    NG = N // 6
    return pl.pallas_call(
        _t4_body,
        grid=(NG // GB,),
        in_specs=[
            pl.BlockSpec((GB, 1536), lambda n: (n, 0)),
            pl.BlockSpec((1536, 512), lambda n: (0, 0)),
            pl.BlockSpec((1, 512), lambda n: (0, 0)),
            pl.BlockSpec((512, 256), lambda n: (0, 0)),
            pl.BlockSpec((1, 256), lambda n: (0, 0)),
            pl.BlockSpec((256, 128), lambda n: (0, 0)),
            pl.BlockSpec((1, 128), lambda n: (0, 0)),
            pl.BlockSpec((128, 7), lambda n: (0, 0)),
            pl.BlockSpec((1, 7), lambda n: (0, 0)),
        ],
        out_specs=pl.BlockSpec((GB, 7), lambda n: (n, 0)),
        out_shape=jax.ShapeDtypeStruct((NG, 7), jnp.float32),
    )(g, f1w, f1b, f2w, f2b, f3w, f3b, pw, pb)


def kernel(x, edge_index, edge_attr, num_graphs, W1, b1, W2, b2, root,
           root_b, gcn_w, gcn_b, f1w, f1b, f2w, f2b, f3w, f3b, pw, pb):
    f32 = jnp.float32
    src = edge_index[0].astype(jnp.int32)
    dst = edge_index[1].astype(jnp.int32)
    src3d_w = src.reshape(NW, EPW // IB, IB)
    src3d = src.reshape(NS, EPT // IBS, IBS)
    dst3d = dst.reshape(NS, EPT // IBS, IBS)
    xpad = jnp.pad(x, ((0, 0), (0, 121)))
    rootpad = jnp.pad(root, ((0, 121), (0, 0)))
    ones8 = jnp.ones((IBS, 8), f32)
    zeros8 = jnp.zeros((NPT, 8), f32)
    zerosr = jnp.zeros((RPT, 128), f32)

    sc_gather_deg, sc_scatter_msg, sc_gcn = _sc_kernels()
    xsrc, deg = sc_gather_deg(xpad, src3d_w, dst3d, ones8, zeros8)
    msg = _t1(edge_attr, xsrc, W1, b1.reshape(1, 128), W2,
              b2.reshape(1, 896))
    agg = sc_scatter_msg(dst3d, msg, zerosr)
    ns = _t2(xpad, agg, deg, rootpad, root_b.reshape(1, 128))
    S = sc_gcn(src3d, dst3d, ns, zerosr)
    h2g = _t3(S, ns, deg, gcn_w, gcn_b.reshape(1, 256))
    g = h2g.reshape(N // 6, 1536)
    return _t4(g, f1w, f1b.reshape(1, 512), f2w, f2b.reshape(1, 256),
               f3w, f3b.reshape(1, 128), pw, pb.reshape(1, 7))


# speedup vs baseline: 1.1540x; 1.0002x over previous
import functools

import jax
import jax.numpy as jnp
from jax import lax
from jax.experimental import pallas as pl
from jax.experimental.pallas import tpu as pltpu
from jax.experimental.pallas import tpu_sc as plsc

N = 8192 * 6
E = 8192 * 5
NC, NS = 2, 16
NW = NC * NS
EPW = E // NW
EPT = E // NS
NPT = N // NS
IB = 128
IBS = 64
NRANGE = 4
NR = N // NRANGE
RPT = NR // NS

_SC_PARAMS = pltpu.CompilerParams(use_tc_tiling_on_sc=False)


@functools.cache
def _sc_kernels():
    mesh = plsc.VectorSubcoreMesh(
        core_axis_name="c", subcore_axis_name="s",
        num_cores=NC, num_subcores=NS)
    g0 = _make_sc_gather_deg(mesh)
    s2 = _make_sc_scatter_msg(mesh)
    s3 = _make_sc_gcn(mesh)
    return g0, s2, s3


def _make_sc_gather_deg(mesh):
    return functools.partial(
        pl.kernel,
        out_type=(jax.ShapeDtypeStruct((E, 128), jnp.float32),
                  jax.ShapeDtypeStruct((N, 128), jnp.float32)),
        mesh=mesh,
        scratch_types=(
            pltpu.VMEM((EPW // IB, IB), jnp.int32),
            pltpu.VMEM((EPT // IBS, IBS), jnp.int32),
            pltpu.VMEM((IB, 128), jnp.float32),
            pltpu.VMEM((IB, 128), jnp.float32),
            pltpu.VMEM((IBS, 8), jnp.float32),
            pltpu.VMEM_SHARED((N, 8), jnp.float32),
            pltpu.SemaphoreType.DMA,
            pltpu.SemaphoreType.DMA,
            pltpu.SemaphoreType.DMA,
            pltpu.SemaphoreType.DMA,
        ),
        compiler_params=_SC_PARAMS,
    )(_sc_gather_deg_body)


def _sc_gather_deg_body(xpad_hbm, src3d_hbm, dst3d_hbm, ones_hbm, zeros8_hbm,
                        xsrc_hbm, deg_hbm,
                        sidx_v, didx_v, rows_v0, rows_v1, ones_v, deg_sp,
                        gsem0, gsem1, wsem0, wsem1):
    cid = lax.axis_index("c")
    sid = lax.axis_index("s")
    wid = sid * NC + cid
    rows_v = (rows_v0, rows_v1)
    gsem = (gsem0, gsem1)
    wsem = (wsem0, wsem1)
    nj = EPW // IB
    pltpu.sync_copy(src3d_hbm.at[wid], sidx_v)
    gd = [None, None]
    wd = [None, None]
    for j in range(nj):
        b = j % 2
        if wd[b] is not None:
            wd[b].wait()
        gd[b] = pltpu.async_copy(xpad_hbm.at[sidx_v.at[j]], rows_v[b],
                                 gsem[b])
        if j > 0:
            pb = (j - 1) % 2
            gd[pb].wait()
            wd[pb] = pltpu.async_copy(
                rows_v[pb],
                xsrc_hbm.at[pl.ds(wid * EPW + (j - 1) * IB, IB)], wsem[pb])
    lb = (nj - 1) % 2
    gd[lb].wait()
    wd[lb] = pltpu.async_copy(
        rows_v[lb], xsrc_hbm.at[pl.ds(wid * EPW + (nj - 1) * IB, IB)],
        wsem[lb])
    wd[0].wait()
    wd[1].wait()

    @pl.when(cid == 0)
    def _deg():
        pltpu.sync_copy(zeros8_hbm, deg_sp.at[pl.ds(sid * NPT, NPT)])
        pltpu.sync_copy(dst3d_hbm.at[sid], didx_v)
        pltpu.sync_copy(ones_hbm, ones_v)
        plsc.subcore_barrier()
        descs = [pltpu.async_copy(ones_v, deg_sp.at[didx_v.at[j]], gsem0,
                                  add=True)
                 for j in range(EPT // IBS)]
        for d in descs:
            d.wait()
        plsc.subcore_barrier()
        pltpu.sync_copy(deg_sp.at[pl.ds(sid * NPT, NPT)],
                        deg_hbm.at[pl.ds(sid * NPT, NPT), pl.ds(0, 8)])


def _make_sc_scatter_msg(mesh):
    return functools.partial(
        pl.kernel,
        out_type=jax.ShapeDtypeStruct((N, 128), jnp.float32),
        mesh=mesh,
        scratch_types=(
            pltpu.VMEM((EPT // IBS, IBS), jnp.int32),
            pltpu.VMEM((EPT // IBS, IBS), jnp.int32),
            pltpu.VMEM((IBS, 128), jnp.float32),
            pltpu.VMEM((IBS, 128), jnp.float32),
            pltpu.VMEM((IBS, 128), jnp.float32),
            pltpu.VMEM_SHARED((NR + 8, 128), jnp.float32),
            pltpu.SemaphoreType.DMA,
            pltpu.SemaphoreType.DMA,
            pltpu.SemaphoreType.DMA,
            pltpu.SemaphoreType.DMA,
            pltpu.SemaphoreType.DMA,
            pltpu.SemaphoreType.DMA,
        ),
        compiler_params=_SC_PARAMS,
    )(_sc_scatter_msg_body)


def _rewrite_range_idx(didx_v, didx_r, lo):
    hi = lo + NR
    for r in range(EPT // IBS):
        for k in range(IBS // 16):
            v = didx_v[r, pl.ds(16 * k, 16)]
            ok = (v >= lo) & (v < hi)
            didx_r[r, pl.ds(16 * k, 16)] = jnp.where(
                ok, v - lo, jnp.full((16,), NR, jnp.int32))


def _sc_scatter_msg_body(dst3d_hbm, msg_hbm, zeros_hbm, agg_hbm,
                         didx_v, didx_r, dat_v0, dat_v1, dat_v2, acc_sp,
                         lsem0, lsem1, lsem2, ssem0, ssem1, ssem2):
    cid = lax.axis_index("c")
    sid = lax.axis_index("s")
    dat_v = (dat_v0, dat_v1, dat_v2)
    lsem = (lsem0, lsem1, lsem2)
    ssem = (ssem0, ssem1, ssem2)
    nj = EPT // IBS
    pltpu.sync_copy(dst3d_hbm.at[sid], didx_v)
    for p in range(2):
        lo = (2 * cid + p) * NR
        _rewrite_range_idx(didx_v, didx_r, lo)
        pltpu.sync_copy(zeros_hbm, acc_sp.at[pl.ds(sid * RPT, RPT)])
        plsc.subcore_barrier()
        ld = [None, None, None]
        sd = [None, None, None]
        ld[0] = pltpu.async_copy(msg_hbm.at[pl.ds(sid * EPT, IBS)], dat_v[0],
                                 lsem[0])
        for j in range(nj):
            b = j % 3
            if j + 1 < nj:
                nb = (j + 1) % 3
                if sd[nb] is not None:
                    sd[nb].wait()
                ld[nb] = pltpu.async_copy(
                    msg_hbm.at[pl.ds(sid * EPT + (j + 1) * IBS, IBS)],
                    dat_v[nb], lsem[nb])
            ld[b].wait()
            sd[b] = pltpu.async_copy(dat_v[b], acc_sp.at[didx_r.at[j]],
                                     ssem[b], add=True)
        for d in sd:
            if d is not None:
                d.wait()
        plsc.subcore_barrier()
        pltpu.sync_copy(acc_sp.at[pl.ds(sid * RPT, RPT)],
                        agg_hbm.at[pl.ds(lo + sid * RPT, RPT)])
        plsc.subcore_barrier()


def _make_sc_gcn(mesh):
    return functools.partial(
        pl.kernel,
        out_type=jax.ShapeDtypeStruct((N, 128), jnp.float32),
        mesh=mesh,
        scratch_types=(
            pltpu.VMEM((EPT // IBS, IBS), jnp.int32),
            pltpu.VMEM((EPT // IBS, IBS), jnp.int32),
            pltpu.VMEM((EPT // IBS, IBS), jnp.int32),
            pltpu.VMEM((IBS, 128), jnp.float32),
            pltpu.VMEM((IBS, 128), jnp.float32),
            pltpu.VMEM((IBS, 128), jnp.float32),
            pltpu.VMEM_SHARED((NR + 8, 128), jnp.float32),
            pltpu.SemaphoreType.DMA,
            pltpu.SemaphoreType.DMA,
            pltpu.SemaphoreType.DMA,
            pltpu.SemaphoreType.DMA,
            pltpu.SemaphoreType.DMA,
            pltpu.SemaphoreType.DMA,
        ),
        compiler_params=_SC_PARAMS,
    )(_sc_gcn_body)


def _sc_gcn_body(src3d_hbm, dst3d_hbm, ns_hbm, zeros_hbm, s_hbm,
                 sidx_v, didx_v, didx_r, dat_v0, dat_v1, dat_v2, acc_sp,
                 lsem0, lsem1, lsem2, ssem0, ssem1, ssem2):
    cid = lax.axis_index("c")
    sid = lax.axis_index("s")
    dat_v = (dat_v0, dat_v1, dat_v2)
    lsem = (lsem0, lsem1, lsem2)
    ssem = (ssem0, ssem1, ssem2)
    nj = EPT // IBS
    pltpu.sync_copy(src3d_hbm.at[sid], sidx_v)
    pltpu.sync_copy(dst3d_hbm.at[sid], didx_v)
    for p in range(2):
        lo = (2 * cid + p) * NR
        _rewrite_range_idx(didx_v, didx_r, lo)
        pltpu.sync_copy(zeros_hbm, acc_sp.at[pl.ds(sid * RPT, RPT)])
        plsc.subcore_barrier()
        ld = [None, None, None]
        sd = [None, None, None]
        ld[0] = pltpu.async_copy(ns_hbm.at[sidx_v.at[0]], dat_v[0], lsem[0])
        for j in range(nj):
            b = j % 3
            if j + 1 < nj:
                nb = (j + 1) % 3
                if sd[nb] is not None:
                    sd[nb].wait()
                ld[nb] = pltpu.async_copy(ns_hbm.at[sidx_v.at[j + 1]],
                                          dat_v[nb], lsem[nb])
            ld[b].wait()
            sd[b] = pltpu.async_copy(dat_v[b], acc_sp.at[didx_r.at[j]],
                                     ssem[b], add=True)
        for d in sd:
            if d is not None:
                d.wait()
        plsc.subcore_barrier()
        pltpu.sync_copy(acc_sp.at[pl.ds(sid * RPT, RPT)],
                        s_hbm.at[pl.ds(lo + sid * RPT, RPT)])
        plsc.subcore_barrier()


EB = 2048


def _t1_body(ea_ref, xs_ref, w1_ref, b1_ref, w2_ref, b2_ref, o_ref):
    bf16 = jnp.bfloat16
    h1 = jnp.maximum(
        jnp.dot(ea_ref[...], w1_ref[...],
                preferred_element_type=jnp.float32) + b1_ref[...], 0.0)
    h1b = h1.astype(bf16)
    w2b = w2_ref[...].astype(bf16)
    xs = xs_ref[...]
    msg = jnp.zeros((EB, 128), jnp.float32)
    for i in range(7):
        h2i = jnp.maximum(
            jnp.dot(h1b, w2b[:, 128 * i:128 * (i + 1)],
                    preferred_element_type=jnp.float32)
            + b2_ref[:, 128 * i:128 * (i + 1)], 0.0)
        msg = msg + xs[:, i:i + 1] * h2i
    o_ref[...] = msg


def _t1(ea, xsrc, W1, b1, W2, b2):
    return pl.pallas_call(
        _t1_body,
        grid=(E // EB,),
        in_specs=[
            pl.BlockSpec((EB, 19), lambda e: (e, 0)),
            pl.BlockSpec((EB, 128), lambda e: (e, 0)),
            pl.BlockSpec((19, 128), lambda e: (0, 0)),
            pl.BlockSpec((1, 128), lambda e: (0, 0)),
            pl.BlockSpec((128, 896), lambda e: (0, 0)),
            pl.BlockSpec((1, 896), lambda e: (0, 0)),
        ],
        out_specs=pl.BlockSpec((EB, 128), lambda e: (e, 0)),
        out_shape=jax.ShapeDtypeStruct((E, 128), jnp.float32),
    )(ea, xsrc, W1, b1, W2, b2)


NB = 4096


def _t2_body(xp_ref, agg_ref, deg_ref, rw_ref, rb_ref, ns_ref):
    node = (jnp.dot(xp_ref[...], rw_ref[...],
                    preferred_element_type=jnp.float32)
            + rb_ref[...] + agg_ref[...])
    dinv = lax.rsqrt(deg_ref[:, 0:1] + 1.0)
    ns_ref[...] = dinv * node


def _t2(xpad, agg, deg, rootpad, root_b):
    f32 = jnp.float32
    return pl.pallas_call(
        _t2_body,
        grid=(N // NB,),
        in_specs=[
            pl.BlockSpec((NB, 128), lambda n: (n, 0)),
            pl.BlockSpec((NB, 128), lambda n: (n, 0)),
            pl.BlockSpec((NB, 128), lambda n: (n, 0)),
            pl.BlockSpec((128, 128), lambda n: (0, 0)),
            pl.BlockSpec((1, 128), lambda n: (0, 0)),
        ],
        out_specs=pl.BlockSpec((NB, 128), lambda n: (n, 0)),
        out_shape=jax.ShapeDtypeStruct((N, 128), f32),
    )(xpad, agg, deg, rootpad, root_b)


def _t3_body(s_ref, ns_ref, deg_ref, gw_ref, gb_ref, out_ref):
    dinv = lax.rsqrt(deg_ref[:, 0:1] + 1.0)
    pre = dinv * (s_ref[...] + ns_ref[...])
    out_ref[...] = jnp.maximum(
        jnp.dot(pre, gw_ref[...], preferred_element_type=jnp.float32)
        + gb_ref[...], 0.0).astype(jnp.bfloat16)


def _t3(S, ns, deg, gcn_w, gcn_b):
    return pl.pallas_call(
        _t3_body,
        grid=(N // NB,),
        in_specs=[
            pl.BlockSpec((NB, 128), lambda n: (n, 0)),
            pl.BlockSpec((NB, 128), lambda n: (n, 0)),
            pl.BlockSpec((NB, 128), lambda n: (n, 0)),
            pl.BlockSpec((128, 256), lambda n: (0, 0)),
            pl.BlockSpec((1, 256), lambda n: (0, 0)),
        ],
        out_specs=pl.BlockSpec((NB, 256), lambda n: (n, 0)),
        out_shape=jax.ShapeDtypeStruct((N, 256), jnp.bfloat16),
    )(S, ns, deg, gcn_w, gcn_b)


GB = 1024


def _t4_body(g_ref, w1, b1, w2, b2, w3, b3, pw, pb, out_ref):
    bf16 = jnp.bfloat16
    t = jnp.maximum(
        jnp.dot(g_ref[...], w1[...].astype(bf16),
                preferred_element_type=jnp.float32) + b1[...], 0.0)
    t = jnp.maximum(
        jnp.dot(t.astype(bf16), w2[...].astype(bf16),
                preferred_element_type=jnp.float32) + b2[...], 0.0)
    t = jnp.maximum(
        jnp.dot(t.astype(bf16), w3[...].astype(bf16),
                preferred_element_type=jnp.float32) + b3[...], 0.0)
    out_ref[...] = (jnp.dot(t, pw[...], preferred_element_type=jnp.float32)
                    + pb[...])


def _t4(g, f1w, f1b, f2w, f2b, f3w, f3b, pw, pb):
    NG = N // 6
    return pl.pallas_call(
        _t4_body,
        grid=(NG // GB,),
        in_specs=[
            pl.BlockSpec((GB, 1536), lambda n: (n, 0)),
            pl.BlockSpec((1536, 512), lambda n: (0, 0)),
            pl.BlockSpec((1, 512), lambda n: (0, 0)),
            pl.BlockSpec((512, 256), lambda n: (0, 0)),
            pl.BlockSpec((1, 256), lambda n: (0, 0)),
            pl.BlockSpec((256, 128), lambda n: (0, 0)),
            pl.BlockSpec((1, 128), lambda n: (0, 0)),
            pl.BlockSpec((128, 7), lambda n: (0, 0)),
            pl.BlockSpec((1, 7), lambda n: (0, 0)),
        ],
        out_specs=pl.BlockSpec((GB, 7), lambda n: (n, 0)),
        out_shape=jax.ShapeDtypeStruct((NG, 7), jnp.float32),
    )(g, f1w, f1b, f2w, f2b, f3w, f3b, pw, pb)


def kernel(x, edge_index, edge_attr, num_graphs, W1, b1, W2, b2, root,
           root_b, gcn_w, gcn_b, f1w, f1b, f2w, f2b, f3w, f3b, pw, pb):
    f32 = jnp.float32
    src = edge_index[0].astype(jnp.int32)
    dst = edge_index[1].astype(jnp.int32)
    src3d_w = src.reshape(NW, EPW // IB, IB)
    src3d = src.reshape(NS, EPT // IBS, IBS)
    dst3d = dst.reshape(NS, EPT // IBS, IBS)
    xpad = jnp.pad(x, ((0, 0), (0, 121)))
    rootpad = jnp.pad(root, ((0, 121), (0, 0)))
    ones8 = jnp.ones((IBS, 8), f32)
    zeros8 = jnp.zeros((NPT, 8), f32)
    zerosr = jnp.zeros((RPT, 128), f32)

    sc_gather_deg, sc_scatter_msg, sc_gcn = _sc_kernels()
    xsrc, deg = sc_gather_deg(xpad, src3d_w, dst3d, ones8, zeros8)
    msg = _t1(edge_attr, xsrc, W1, b1.reshape(1, 128), W2,
              b2.reshape(1, 896))
    agg = sc_scatter_msg(dst3d, msg, zerosr)
    ns = _t2(xpad, agg, deg, rootpad, root_b.reshape(1, 128))
    S = sc_gcn(src3d, dst3d, ns, zerosr)
    h2g = _t3(S, ns, deg, gcn_w, gcn_b.reshape(1, 256))
    g = h2g.reshape(N // 6, 1536)
    return _t4(g, f1w, f1b.reshape(1, 512), f2w, f2b.reshape(1, 256),
               f3w, f3b.reshape(1, 128), pw, pb.reshape(1, 7))


# speedup vs baseline: 1.2341x; 1.0694x over previous
import functools

import jax
import jax.numpy as jnp
from jax import lax
from jax.experimental import pallas as pl
from jax.experimental.pallas import tpu as pltpu
from jax.experimental.pallas import tpu_sc as plsc

N = 8192 * 6
E = 8192 * 5
NC, NS = 2, 16
NW = NC * NS
EPW = E // NW
EPT = E // NS
NPT = N // NS
IB = 128
IBS = 64
NRANGE = 4
NR = N // NRANGE
RPT = NR // NS

_SC_PARAMS = pltpu.CompilerParams(use_tc_tiling_on_sc=False)


@functools.cache
def _sc_kernels():
    mesh = plsc.VectorSubcoreMesh(
        core_axis_name="c", subcore_axis_name="s",
        num_cores=NC, num_subcores=NS)
    g0 = _make_sc_gather_deg(mesh)
    s2 = _make_sc_scatter_msg(mesh)
    s3 = _make_sc_gcn(mesh)
    return g0, s2, s3


def _make_sc_gather_deg(mesh):
    return functools.partial(
        pl.kernel,
        out_type=(jax.ShapeDtypeStruct((E, 128), jnp.float32),
                  jax.ShapeDtypeStruct((N, 128), jnp.float32)),
        mesh=mesh,
        scratch_types=(
            pltpu.VMEM((EPW // IB, IB), jnp.int32),
            pltpu.VMEM((EPT // IBS, IBS), jnp.int32),
            pltpu.VMEM((IB, 128), jnp.float32),
            pltpu.VMEM((IB, 128), jnp.float32),
            pltpu.VMEM((IBS, 8), jnp.float32),
            pltpu.VMEM_SHARED((N, 8), jnp.float32),
            pltpu.SemaphoreType.DMA,
            pltpu.SemaphoreType.DMA,
            pltpu.SemaphoreType.DMA,
            pltpu.SemaphoreType.DMA,
        ),
        compiler_params=_SC_PARAMS,
    )(_sc_gather_deg_body)


def _sc_gather_deg_body(xpad_hbm, src3d_hbm, dst3d_hbm, ones_hbm, zeros8_hbm,
                        xsrc_hbm, deg_hbm,
                        sidx_v, didx_v, rows_v0, rows_v1, ones_v, deg_sp,
                        gsem0, gsem1, wsem0, wsem1):
    cid = lax.axis_index("c")
    sid = lax.axis_index("s")
    wid = sid * NC + cid
    rows_v = (rows_v0, rows_v1)
    gsem = (gsem0, gsem1)
    wsem = (wsem0, wsem1)
    nj = EPW // IB
    pltpu.sync_copy(src3d_hbm.at[wid], sidx_v)
    gd = [None, None]
    wd = [None, None]
    for j in range(nj):
        b = j % 2
        if wd[b] is not None:
            wd[b].wait()
        gd[b] = pltpu.async_copy(xpad_hbm.at[sidx_v.at[j]], rows_v[b],
                                 gsem[b])
        if j > 0:
            pb = (j - 1) % 2
            gd[pb].wait()
            wd[pb] = pltpu.async_copy(
                rows_v[pb],
                xsrc_hbm.at[pl.ds(wid * EPW + (j - 1) * IB, IB)], wsem[pb])
    lb = (nj - 1) % 2
    gd[lb].wait()
    wd[lb] = pltpu.async_copy(
        rows_v[lb], xsrc_hbm.at[pl.ds(wid * EPW + (nj - 1) * IB, IB)],
        wsem[lb])
    wd[0].wait()
    wd[1].wait()

    @pl.when(cid == 0)
    def _deg():
        pltpu.sync_copy(zeros8_hbm, deg_sp.at[pl.ds(sid * NPT, NPT)])
        pltpu.sync_copy(dst3d_hbm.at[sid], didx_v)
        pltpu.sync_copy(ones_hbm, ones_v)
        plsc.subcore_barrier()
        descs = [pltpu.async_copy(ones_v, deg_sp.at[didx_v.at[j]], gsem0,
                                  add=True)
                 for j in range(EPT // IBS)]
        for d in descs:
            d.wait()
        plsc.subcore_barrier()
        pltpu.sync_copy(deg_sp.at[pl.ds(sid * NPT, NPT)],
                        deg_hbm.at[pl.ds(sid * NPT, NPT), pl.ds(0, 8)])


def _make_sc_scatter_msg(mesh):
    return functools.partial(
        pl.kernel,
        out_type=jax.ShapeDtypeStruct((N, 128), jnp.float32),
        mesh=mesh,
        scratch_types=(
            pltpu.VMEM((EPT // IBS, IBS), jnp.int32),
            pltpu.VMEM((EPT // IBS, IBS), jnp.int32),
            pltpu.VMEM((IBS, 128), jnp.float32),
            pltpu.VMEM((IBS, 128), jnp.float32),
            pltpu.VMEM((IBS, 128), jnp.float32),
            pltpu.VMEM_SHARED((NR + 8, 128), jnp.float32),
            pltpu.SemaphoreType.DMA,
            pltpu.SemaphoreType.DMA,
            pltpu.SemaphoreType.DMA,
            pltpu.SemaphoreType.DMA,
            pltpu.SemaphoreType.DMA,
            pltpu.SemaphoreType.DMA,
        ),
        compiler_params=_SC_PARAMS,
    )(_sc_scatter_msg_body)


def _rewrite_range_idx(didx_v, didx_r, lo):
    hi = lo + NR
    for r in range(EPT // IBS):
        for k in range(IBS // 16):
            v = didx_v[r, pl.ds(16 * k, 16)]
            ok = (v >= lo) & (v < hi)
            didx_r[r, pl.ds(16 * k, 16)] = jnp.where(
                ok, v - lo, jnp.full((16,), NR, jnp.int32))


def _sc_scatter_msg_body(dst3d_hbm, msg_hbm, zeros_hbm, agg_hbm,
                         didx_v, didx_r, dat_v0, dat_v1, dat_v2, acc_sp,
                         lsem0, lsem1, lsem2, ssem0, ssem1, ssem2):
    cid = lax.axis_index("c")
    sid = lax.axis_index("s")
    dat_v = (dat_v0, dat_v1, dat_v2)
    lsem = (lsem0, lsem1, lsem2)
    ssem = (ssem0, ssem1, ssem2)
    nj = EPT // IBS
    pltpu.sync_copy(dst3d_hbm.at[sid], didx_v)
    for p in range(2):
        lo = (2 * cid + p) * NR
        _rewrite_range_idx(didx_v, didx_r, lo)
        pltpu.sync_copy(zeros_hbm, acc_sp.at[pl.ds(sid * RPT, RPT)])
        plsc.subcore_barrier()
        ld = [None, None, None]
        sd = [None, None, None]
        ld[0] = pltpu.async_copy(msg_hbm.at[pl.ds(sid * EPT, IBS)], dat_v[0],
                                 lsem[0])
        for j in range(nj):
            b = j % 3
            if j + 1 < nj:
                nb = (j + 1) % 3
                if sd[nb] is not None:
                    sd[nb].wait()
                ld[nb] = pltpu.async_copy(
                    msg_hbm.at[pl.ds(sid * EPT + (j + 1) * IBS, IBS)],
                    dat_v[nb], lsem[nb])
            ld[b].wait()
            sd[b] = pltpu.async_copy(dat_v[b], acc_sp.at[didx_r.at[j]],
                                     ssem[b], add=True)
        for d in sd:
            if d is not None:
                d.wait()
        plsc.subcore_barrier()
        pltpu.sync_copy(acc_sp.at[pl.ds(sid * RPT, RPT)],
                        agg_hbm.at[pl.ds(lo + sid * RPT, RPT)])
        plsc.subcore_barrier()


def _make_sc_gcn(mesh):
    return functools.partial(
        pl.kernel,
        out_type=jax.ShapeDtypeStruct((N, 128), jnp.float32),
        mesh=mesh,
        scratch_types=(
            pltpu.VMEM((EPT // IBS, IBS), jnp.int32),
            pltpu.VMEM((EPT // IBS, IBS), jnp.int32),
            pltpu.VMEM((EPT // IBS, IBS), jnp.int32),
            pltpu.VMEM((IBS, 128), jnp.float32),
            pltpu.VMEM((IBS, 128), jnp.float32),
            pltpu.VMEM((IBS, 128), jnp.float32),
            pltpu.VMEM_SHARED((NR + 8, 128), jnp.float32),
            pltpu.SemaphoreType.DMA,
            pltpu.SemaphoreType.DMA,
            pltpu.SemaphoreType.DMA,
            pltpu.SemaphoreType.DMA,
            pltpu.SemaphoreType.DMA,
            pltpu.SemaphoreType.DMA,
        ),
        compiler_params=_SC_PARAMS,
    )(_sc_gcn_body)


def _sc_gcn_body(src3d_hbm, dst3d_hbm, ns_hbm, zeros_hbm, s_hbm,
                 sidx_v, didx_v, didx_r, dat_v0, dat_v1, dat_v2, acc_sp,
                 lsem0, lsem1, lsem2, ssem0, ssem1, ssem2):
    cid = lax.axis_index("c")
    sid = lax.axis_index("s")
    dat_v = (dat_v0, dat_v1, dat_v2)
    lsem = (lsem0, lsem1, lsem2)
    ssem = (ssem0, ssem1, ssem2)
    nj = EPT // IBS
    pltpu.sync_copy(src3d_hbm.at[sid], sidx_v)
    pltpu.sync_copy(dst3d_hbm.at[sid], didx_v)
    for p in range(2):
        lo = (2 * cid + p) * NR
        _rewrite_range_idx(didx_v, didx_r, lo)
        pltpu.sync_copy(zeros_hbm, acc_sp.at[pl.ds(sid * RPT, RPT)])
        plsc.subcore_barrier()
        ld = [None, None, None]
        sd = [None, None, None]
        ld[0] = pltpu.async_copy(ns_hbm.at[sidx_v.at[0]], dat_v[0], lsem[0])
        for j in range(nj):
            b = j % 3
            if j + 1 < nj:
                nb = (j + 1) % 3
                if sd[nb] is not None:
                    sd[nb].wait()
                ld[nb] = pltpu.async_copy(ns_hbm.at[sidx_v.at[j + 1]],
                                          dat_v[nb], lsem[nb])
            ld[b].wait()
            sd[b] = pltpu.async_copy(dat_v[b], acc_sp.at[didx_r.at[j]],
                                     ssem[b], add=True)
        for d in sd:
            if d is not None:
                d.wait()
        plsc.subcore_barrier()
        pltpu.sync_copy(acc_sp.at[pl.ds(sid * RPT, RPT)],
                        s_hbm.at[pl.ds(lo + sid * RPT, RPT)])
        plsc.subcore_barrier()


EB = 2048


def _t1_body(ea_ref, xs_ref, w1_ref, b1_ref, w2_ref, b2_ref, o_ref):
    bf16 = jnp.bfloat16
    h1 = jnp.maximum(
        jnp.dot(ea_ref[...], w1_ref[...],
                preferred_element_type=jnp.float32) + b1_ref[...], 0.0)
    h1b = h1.astype(bf16)
    w2b = w2_ref[...].astype(bf16)
    xs = xs_ref[...]
    msg = jnp.zeros((EB, 128), jnp.float32)
    for i in range(7):
        h2i = jnp.maximum(
            jnp.dot(h1b, w2b[:, 128 * i:128 * (i + 1)],
                    preferred_element_type=jnp.float32)
            + b2_ref[:, 128 * i:128 * (i + 1)], 0.0)
        msg = msg + xs[:, i:i + 1] * h2i
    o_ref[...] = msg


def _t1(ea, xsrc, W1, b1, W2, b2):
    return pl.pallas_call(
        _t1_body,
        grid=(E // EB,),
        in_specs=[
            pl.BlockSpec((EB, 19), lambda e: (e, 0)),
            pl.BlockSpec((EB, 128), lambda e: (e, 0)),
            pl.BlockSpec((19, 128), lambda e: (0, 0)),
            pl.BlockSpec((1, 128), lambda e: (0, 0)),
            pl.BlockSpec((128, 896), lambda e: (0, 0)),
            pl.BlockSpec((1, 896), lambda e: (0, 0)),
        ],
        out_specs=pl.BlockSpec((EB, 128), lambda e: (e, 0)),
        out_shape=jax.ShapeDtypeStruct((E, 128), jnp.float32),
    )(ea, xsrc, W1, b1, W2, b2)


NB = 4096


def _t2_body(xp_ref, agg_ref, deg_ref, rw_ref, rb_ref, ns_ref):
    node = (jnp.dot(xp_ref[...], rw_ref[...],
                    preferred_element_type=jnp.float32)
            + rb_ref[...] + agg_ref[...])
    dinv = lax.rsqrt(deg_ref[:, 0:1] + 1.0)
    ns_ref[...] = dinv * node


def _t2(xpad, agg, deg, rootpad, root_b):
    f32 = jnp.float32
    return pl.pallas_call(
        _t2_body,
        grid=(N // NB,),
        in_specs=[
            pl.BlockSpec((NB, 128), lambda n: (n, 0)),
            pl.BlockSpec((NB, 128), lambda n: (n, 0)),
            pl.BlockSpec((NB, 128), lambda n: (n, 0)),
            pl.BlockSpec((128, 128), lambda n: (0, 0)),
            pl.BlockSpec((1, 128), lambda n: (0, 0)),
        ],
        out_specs=pl.BlockSpec((NB, 128), lambda n: (n, 0)),
        out_shape=jax.ShapeDtypeStruct((N, 128), f32),
    )(xpad, agg, deg, rootpad, root_b)


GB = 512


def _t34_body(s_ref, ns_ref, deg_ref, gw_ref, gb_ref,
              w1, b1, w2, b2, w3, b3, pw, pb, out_ref):
    bf16 = jnp.bfloat16
    dinv = lax.rsqrt(deg_ref[:, 0:1] + 1.0)
    pre = dinv * (s_ref[...] + ns_ref[...])
    h2 = jnp.maximum(
        jnp.dot(pre, gw_ref[...], preferred_element_type=jnp.float32)
        + gb_ref[...], 0.0)
    h23 = h2.reshape(GB, 6, 256).astype(bf16)
    w1b = w1[...].astype(bf16)
    t = jnp.zeros((GB, 512), jnp.float32)
    for j in range(6):
        t = t + jnp.dot(h23[:, j, :], w1b[256 * j:256 * (j + 1), :],
                        preferred_element_type=jnp.float32)
    t = jnp.maximum(t + b1[...], 0.0)
    t = jnp.maximum(
        jnp.dot(t.astype(bf16), w2[...].astype(bf16),
                preferred_element_type=jnp.float32) + b2[...], 0.0)
    t = jnp.maximum(
        jnp.dot(t.astype(bf16), w3[...].astype(bf16),
                preferred_element_type=jnp.float32) + b3[...], 0.0)
    out_ref[...] = (jnp.dot(t, pw[...], preferred_element_type=jnp.float32)
                    + pb[...])


def _t34(S, ns, deg, gcn_w, gcn_b, f1w, f1b, f2w, f2b, f3w, f3b, pw, pb):
    NG = N // 6
    NBR = 6 * GB
    return pl.pallas_call(
        _t34_body,
        grid=(NG // GB,),
        in_specs=[
            pl.BlockSpec((NBR, 128), lambda n: (n, 0)),
            pl.BlockSpec((NBR, 128), lambda n: (n, 0)),
            pl.BlockSpec((NBR, 128), lambda n: (n, 0)),
            pl.BlockSpec((128, 256), lambda n: (0, 0)),
            pl.BlockSpec((1, 256), lambda n: (0, 0)),
            pl.BlockSpec((1536, 512), lambda n: (0, 0)),
            pl.BlockSpec((1, 512), lambda n: (0, 0)),
            pl.BlockSpec((512, 256), lambda n: (0, 0)),
            pl.BlockSpec((1, 256), lambda n: (0, 0)),
            pl.BlockSpec((256, 128), lambda n: (0, 0)),
            pl.BlockSpec((1, 128), lambda n: (0, 0)),
            pl.BlockSpec((128, 7), lambda n: (0, 0)),
            pl.BlockSpec((1, 7), lambda n: (0, 0)),
        ],
        out_specs=pl.BlockSpec((GB, 7), lambda n: (n, 0)),
        out_shape=jax.ShapeDtypeStruct((NG, 7), jnp.float32),
    )(S, ns, deg, gcn_w, gcn_b, f1w, f1b, f2w, f2b, f3w, f3b, pw, pb)


def kernel(x, edge_index, edge_attr, num_graphs, W1, b1, W2, b2, root,
           root_b, gcn_w, gcn_b, f1w, f1b, f2w, f2b, f3w, f3b, pw, pb):
    f32 = jnp.float32
    src = edge_index[0].astype(jnp.int32)
    dst = edge_index[1].astype(jnp.int32)
    src3d_w = src.reshape(NW, EPW // IB, IB)
    src3d = src.reshape(NS, EPT // IBS, IBS)
    dst3d = dst.reshape(NS, EPT // IBS, IBS)
    xpad = jnp.pad(x, ((0, 0), (0, 121)))
    rootpad = jnp.pad(root, ((0, 121), (0, 0)))
    ones8 = jnp.ones((IBS, 8), f32)
    zeros8 = jnp.zeros((NPT, 8), f32)
    zerosr = jnp.zeros((RPT, 128), f32)

    sc_gather_deg, sc_scatter_msg, sc_gcn = _sc_kernels()
    xsrc, deg = sc_gather_deg(xpad, src3d_w, dst3d, ones8, zeros8)
    msg = _t1(edge_attr, xsrc, W1, b1.reshape(1, 128), W2,
              b2.reshape(1, 896))
    agg = sc_scatter_msg(dst3d, msg, zerosr)
    ns = _t2(xpad, agg, deg, rootpad, root_b.reshape(1, 128))
    S = sc_gcn(src3d, dst3d, ns, zerosr)
    return _t34(S, ns, deg, gcn_w, gcn_b.reshape(1, 256),
                f1w, f1b.reshape(1, 512), f2w, f2b.reshape(1, 256),
                f3w, f3b.reshape(1, 128), pw, pb.reshape(1, 7))


# speedup vs baseline: 1.3196x; 1.0692x over previous
import functools

import jax
import jax.numpy as jnp
from jax import lax
from jax.experimental import pallas as pl
from jax.experimental.pallas import tpu as pltpu
from jax.experimental.pallas import tpu_sc as plsc

N = 8192 * 6
E = 8192 * 5
NC, NS = 2, 16
NW = NC * NS
EPW = E // NW
EPT = E // NS
NPT = N // NS
IB = 128
IBS = 64
NRANGE = 4
NR = N // NRANGE
RPT = NR // NS

_SC_PARAMS = pltpu.CompilerParams(use_tc_tiling_on_sc=False)


@functools.cache
def _sc_kernels():
    mesh = plsc.VectorSubcoreMesh(
        core_axis_name="c", subcore_axis_name="s",
        num_cores=NC, num_subcores=NS)
    g0 = _make_sc_gather_deg(mesh)
    s2 = _make_sc_scatter_msg(mesh)
    s3 = _make_sc_gcn(mesh)
    return g0, s2, s3


def _make_sc_gather_deg(mesh):
    return functools.partial(
        pl.kernel,
        out_type=(jax.ShapeDtypeStruct((E, 128), jnp.float32),
                  jax.ShapeDtypeStruct((N, 128), jnp.float32)),
        mesh=mesh,
        scratch_types=(
            pltpu.VMEM((EPW // IB, IB), jnp.int32),
            pltpu.VMEM((EPT // IBS, IBS), jnp.int32),
            pltpu.VMEM((IB, 128), jnp.float32),
            pltpu.VMEM((IB, 128), jnp.float32),
            pltpu.VMEM((IBS, 8), jnp.float32),
            pltpu.VMEM_SHARED((N, 8), jnp.float32),
            pltpu.SemaphoreType.DMA,
            pltpu.SemaphoreType.DMA,
            pltpu.SemaphoreType.DMA,
            pltpu.SemaphoreType.DMA,
        ),
        compiler_params=_SC_PARAMS,
    )(_sc_gather_deg_body)


def _sc_gather_deg_body(xpad_hbm, src3d_hbm, dst3d_hbm, ones_hbm, zeros8_hbm,
                        xsrc_hbm, deg_hbm,
                        sidx_v, didx_v, rows_v0, rows_v1, ones_v, deg_sp,
                        gsem0, gsem1, wsem0, wsem1):
    cid = lax.axis_index("c")
    sid = lax.axis_index("s")
    wid = sid * NC + cid
    rows_v = (rows_v0, rows_v1)
    gsem = (gsem0, gsem1)
    wsem = (wsem0, wsem1)
    nj = EPW // IB
    pltpu.sync_copy(src3d_hbm.at[wid], sidx_v)
    gd = [None, None]
    wd = [None, None]
    for j in range(nj):
        b = j % 2
        if wd[b] is not None:
            wd[b].wait()
        gd[b] = pltpu.async_copy(xpad_hbm.at[sidx_v.at[j]], rows_v[b],
                                 gsem[b])
        if j > 0:
            pb = (j - 1) % 2
            gd[pb].wait()
            wd[pb] = pltpu.async_copy(
                rows_v[pb],
                xsrc_hbm.at[pl.ds(wid * EPW + (j - 1) * IB, IB)], wsem[pb])
    lb = (nj - 1) % 2
    gd[lb].wait()
    wd[lb] = pltpu.async_copy(
        rows_v[lb], xsrc_hbm.at[pl.ds(wid * EPW + (nj - 1) * IB, IB)],
        wsem[lb])
    wd[0].wait()
    wd[1].wait()

    @pl.when(cid == 0)
    def _deg():
        pltpu.sync_copy(zeros8_hbm, deg_sp.at[pl.ds(sid * NPT, NPT)])
        pltpu.sync_copy(dst3d_hbm.at[sid], didx_v)
        pltpu.sync_copy(ones_hbm, ones_v)
        plsc.subcore_barrier()
        descs = [pltpu.async_copy(ones_v, deg_sp.at[didx_v.at[j]], gsem0,
                                  add=True)
                 for j in range(EPT // IBS)]
        for d in descs:
            d.wait()
        plsc.subcore_barrier()
        pltpu.sync_copy(deg_sp.at[pl.ds(sid * NPT, NPT)],
                        deg_hbm.at[pl.ds(sid * NPT, NPT), pl.ds(0, 8)])


def _make_sc_scatter_msg(mesh):
    return functools.partial(
        pl.kernel,
        out_type=jax.ShapeDtypeStruct((N, 128), jnp.float32),
        mesh=mesh,
        scratch_types=(
            pltpu.VMEM((EPT // IBS, IBS), jnp.int32),
            pltpu.VMEM((IBS, 32), jnp.float32),
            pltpu.VMEM((IBS, 32), jnp.float32),
            pltpu.VMEM((IBS, 32), jnp.float32),
            pltpu.VMEM_SHARED((N, 32), jnp.float32),
            pltpu.SemaphoreType.DMA,
            pltpu.SemaphoreType.DMA,
            pltpu.SemaphoreType.DMA,
            pltpu.SemaphoreType.DMA,
            pltpu.SemaphoreType.DMA,
            pltpu.SemaphoreType.DMA,
        ),
        compiler_params=_SC_PARAMS,
    )(_sc_scatter_msg_body)


def _sc_scatter_msg_body(dst3d_hbm, msg_hbm, zeros_hbm, agg_hbm,
                         didx_v, dat_v0, dat_v1, dat_v2, acc_sp,
                         lsem0, lsem1, lsem2, ssem0, ssem1, ssem2):
    cid = lax.axis_index("c")
    sid = lax.axis_index("s")
    dat_v = (dat_v0, dat_v1, dat_v2)
    lsem = (lsem0, lsem1, lsem2)
    ssem = (ssem0, ssem1, ssem2)
    nj = EPT // IBS
    pltpu.sync_copy(dst3d_hbm.at[sid], didx_v)
    for p in range(2):
        col = cid * 64 + p * 32
        pltpu.sync_copy(zeros_hbm, acc_sp.at[pl.ds(sid * NPT, NPT)])
        plsc.subcore_barrier()
        ld = [None, None, None]
        sd = [None, None, None]
        ld[0] = pltpu.async_copy(
            msg_hbm.at[pl.ds(sid * EPT, IBS), pl.ds(col, 32)], dat_v[0],
            lsem[0])
        for j in range(nj):
            b = j % 3
            if j + 1 < nj:
                nb = (j + 1) % 3
                if sd[nb] is not None:
                    sd[nb].wait()
                ld[nb] = pltpu.async_copy(
                    msg_hbm.at[pl.ds(sid * EPT + (j + 1) * IBS, IBS),
                               pl.ds(col, 32)], dat_v[nb], lsem[nb])
            ld[b].wait()
            sd[b] = pltpu.async_copy(dat_v[b], acc_sp.at[didx_v.at[j]],
                                     ssem[b], add=True)
        for d in sd:
            if d is not None:
                d.wait()
        plsc.subcore_barrier()
        pltpu.sync_copy(acc_sp.at[pl.ds(sid * NPT, NPT)],
                        agg_hbm.at[pl.ds(sid * NPT, NPT), pl.ds(col, 32)])
        plsc.subcore_barrier()


def _rewrite_range_idx(didx_v, didx_r, lo):
    hi = lo + NR
    for r in range(EPT // IBS):
        for k in range(IBS // 16):
            v = didx_v[r, pl.ds(16 * k, 16)]
            ok = (v >= lo) & (v < hi)
            didx_r[r, pl.ds(16 * k, 16)] = jnp.where(
                ok, v - lo, jnp.full((16,), NR, jnp.int32))


def _make_sc_gcn(mesh):
    return functools.partial(
        pl.kernel,
        out_type=jax.ShapeDtypeStruct((N, 128), jnp.float32),
        mesh=mesh,
        scratch_types=(
            pltpu.VMEM((EPT // IBS, IBS), jnp.int32),
            pltpu.VMEM((EPT // IBS, IBS), jnp.int32),
            pltpu.VMEM((EPT // IBS, IBS), jnp.int32),
            pltpu.VMEM((IBS, 128), jnp.float32),
            pltpu.VMEM((IBS, 128), jnp.float32),
            pltpu.VMEM((IBS, 128), jnp.float32),
            pltpu.VMEM_SHARED((NR + 8, 128), jnp.float32),
            pltpu.SemaphoreType.DMA,
            pltpu.SemaphoreType.DMA,
            pltpu.SemaphoreType.DMA,
            pltpu.SemaphoreType.DMA,
            pltpu.SemaphoreType.DMA,
            pltpu.SemaphoreType.DMA,
        ),
        compiler_params=_SC_PARAMS,
    )(_sc_gcn_body)


def _sc_gcn_body(src3d_hbm, dst3d_hbm, ns_hbm, zeros_hbm, s_hbm,
                 sidx_v, didx_v, didx_r, dat_v0, dat_v1, dat_v2, acc_sp,
                 lsem0, lsem1, lsem2, ssem0, ssem1, ssem2):
    cid = lax.axis_index("c")
    sid = lax.axis_index("s")
    dat_v = (dat_v0, dat_v1, dat_v2)
    lsem = (lsem0, lsem1, lsem2)
    ssem = (ssem0, ssem1, ssem2)
    nj = EPT // IBS
    pltpu.sync_copy(src3d_hbm.at[sid], sidx_v)
    pltpu.sync_copy(dst3d_hbm.at[sid], didx_v)
    for p in range(2):
        lo = (2 * cid + p) * NR
        _rewrite_range_idx(didx_v, didx_r, lo)
        pltpu.sync_copy(zeros_hbm, acc_sp.at[pl.ds(sid * RPT, RPT)])
        plsc.subcore_barrier()
        ld = [None, None, None]
        sd = [None, None, None]
        ld[0] = pltpu.async_copy(ns_hbm.at[sidx_v.at[0]], dat_v[0], lsem[0])
        for j in range(nj):
            b = j % 3
            if j + 1 < nj:
                nb = (j + 1) % 3
                if sd[nb] is not None:
                    sd[nb].wait()
                ld[nb] = pltpu.async_copy(ns_hbm.at[sidx_v.at[j + 1]],
                                          dat_v[nb], lsem[nb])
            ld[b].wait()
            sd[b] = pltpu.async_copy(dat_v[b], acc_sp.at[didx_r.at[j]],
                                     ssem[b], add=True)
        for d in sd:
            if d is not None:
                d.wait()
        plsc.subcore_barrier()
        pltpu.sync_copy(acc_sp.at[pl.ds(sid * RPT, RPT)],
                        s_hbm.at[pl.ds(lo + sid * RPT, RPT)])
        plsc.subcore_barrier()


EB = 2048


def _t1_body(ea_ref, xs_ref, w1_ref, b1_ref, w2_ref, b2_ref, o_ref):
    bf16 = jnp.bfloat16
    h1 = jnp.maximum(
        jnp.dot(ea_ref[...], w1_ref[...],
                preferred_element_type=jnp.float32) + b1_ref[...], 0.0)
    h1b = h1.astype(bf16)
    w2b = w2_ref[...].astype(bf16)
    xs = xs_ref[...]
    msg = jnp.zeros((EB, 128), jnp.float32)
    for i in range(7):
        h2i = jnp.maximum(
            jnp.dot(h1b, w2b[:, 128 * i:128 * (i + 1)],
                    preferred_element_type=jnp.float32)
            + b2_ref[:, 128 * i:128 * (i + 1)], 0.0)
        msg = msg + xs[:, i:i + 1] * h2i
    o_ref[...] = msg


def _t1(ea, xsrc, W1, b1, W2, b2):
    return pl.pallas_call(
        _t1_body,
        grid=(E // EB,),
        in_specs=[
            pl.BlockSpec((EB, 19), lambda e: (e, 0)),
            pl.BlockSpec((EB, 128), lambda e: (e, 0)),
            pl.BlockSpec((19, 128), lambda e: (0, 0)),
            pl.BlockSpec((1, 128), lambda e: (0, 0)),
            pl.BlockSpec((128, 896), lambda e: (0, 0)),
            pl.BlockSpec((1, 896), lambda e: (0, 0)),
        ],
        out_specs=pl.BlockSpec((EB, 128), lambda e: (e, 0)),
        out_shape=jax.ShapeDtypeStruct((E, 128), jnp.float32),
    )(ea, xsrc, W1, b1, W2, b2)


NB = 4096


def _t2_body(xp_ref, agg_ref, deg_ref, rw_ref, rb_ref, ns_ref):
    node = (jnp.dot(xp_ref[...], rw_ref[...],
                    preferred_element_type=jnp.float32)
            + rb_ref[...] + agg_ref[...])
    dinv = lax.rsqrt(deg_ref[:, 0:1] + 1.0)
    ns_ref[...] = dinv * node


def _t2(xpad, agg, deg, rootpad, root_b):
    f32 = jnp.float32
    return pl.pallas_call(
        _t2_body,
        grid=(N // NB,),
        in_specs=[
            pl.BlockSpec((NB, 128), lambda n: (n, 0)),
            pl.BlockSpec((NB, 128), lambda n: (n, 0)),
            pl.BlockSpec((NB, 128), lambda n: (n, 0)),
            pl.BlockSpec((128, 128), lambda n: (0, 0)),
            pl.BlockSpec((1, 128), lambda n: (0, 0)),
        ],
        out_specs=pl.BlockSpec((NB, 128), lambda n: (n, 0)),
        out_shape=jax.ShapeDtypeStruct((N, 128), f32),
    )(xpad, agg, deg, rootpad, root_b)


GB = 512


def _t34_body(s_ref, ns_ref, deg_ref, gw_ref, gb_ref,
              w1, b1, w2, b2, w3, b3, pw, pb, out_ref):
    bf16 = jnp.bfloat16
    dinv = lax.rsqrt(deg_ref[:, 0:1] + 1.0)
    pre = dinv * (s_ref[...] + ns_ref[...])
    h2 = jnp.maximum(
        jnp.dot(pre, gw_ref[...], preferred_element_type=jnp.float32)
        + gb_ref[...], 0.0)
    h23 = h2.reshape(GB, 6, 256).astype(bf16)
    w1b = w1[...].astype(bf16)
    t = jnp.zeros((GB, 512), jnp.float32)
    for j in range(6):
        t = t + jnp.dot(h23[:, j, :], w1b[256 * j:256 * (j + 1), :],
                        preferred_element_type=jnp.float32)
    t = jnp.maximum(t + b1[...], 0.0)
    t = jnp.maximum(
        jnp.dot(t.astype(bf16), w2[...].astype(bf16),
                preferred_element_type=jnp.float32) + b2[...], 0.0)
    t = jnp.maximum(
        jnp.dot(t.astype(bf16), w3[...].astype(bf16),
                preferred_element_type=jnp.float32) + b3[...], 0.0)
    out_ref[...] = (jnp.dot(t, pw[...], preferred_element_type=jnp.float32)
                    + pb[...])


def _t34(S, ns, deg, gcn_w, gcn_b, f1w, f1b, f2w, f2b, f3w, f3b, pw, pb):
    NG = N // 6
    NBR = 6 * GB
    return pl.pallas_call(
        _t34_body,
        grid=(NG // GB,),
        in_specs=[
            pl.BlockSpec((NBR, 128), lambda n: (n, 0)),
            pl.BlockSpec((NBR, 128), lambda n: (n, 0)),
            pl.BlockSpec((NBR, 128), lambda n: (n, 0)),
            pl.BlockSpec((128, 256), lambda n: (0, 0)),
            pl.BlockSpec((1, 256), lambda n: (0, 0)),
            pl.BlockSpec((1536, 512), lambda n: (0, 0)),
            pl.BlockSpec((1, 512), lambda n: (0, 0)),
            pl.BlockSpec((512, 256), lambda n: (0, 0)),
            pl.BlockSpec((1, 256), lambda n: (0, 0)),
            pl.BlockSpec((256, 128), lambda n: (0, 0)),
            pl.BlockSpec((1, 128), lambda n: (0, 0)),
            pl.BlockSpec((128, 7), lambda n: (0, 0)),
            pl.BlockSpec((1, 7), lambda n: (0, 0)),
        ],
        out_specs=pl.BlockSpec((GB, 7), lambda n: (n, 0)),
        out_shape=jax.ShapeDtypeStruct((NG, 7), jnp.float32),
    )(S, ns, deg, gcn_w, gcn_b, f1w, f1b, f2w, f2b, f3w, f3b, pw, pb)


def kernel(x, edge_index, edge_attr, num_graphs, W1, b1, W2, b2, root,
           root_b, gcn_w, gcn_b, f1w, f1b, f2w, f2b, f3w, f3b, pw, pb):
    f32 = jnp.float32
    src = edge_index[0].astype(jnp.int32)
    dst = edge_index[1].astype(jnp.int32)
    src3d_w = src.reshape(NW, EPW // IB, IB)
    src3d = src.reshape(NS, EPT // IBS, IBS)
    dst3d = dst.reshape(NS, EPT // IBS, IBS)
    xpad = jnp.pad(x, ((0, 0), (0, 121)))
    rootpad = jnp.pad(root, ((0, 121), (0, 0)))
    ones8 = jnp.ones((IBS, 8), f32)
    zeros8 = jnp.zeros((NPT, 8), f32)
    zerosr = jnp.zeros((RPT, 128), f32)
    zerosn = jnp.zeros((NPT, 32), f32)

    sc_gather_deg, sc_scatter_msg, sc_gcn = _sc_kernels()
    xsrc, deg = sc_gather_deg(xpad, src3d_w, dst3d, ones8, zeros8)
    msg = _t1(edge_attr, xsrc, W1, b1.reshape(1, 128), W2,
              b2.reshape(1, 896))
    agg = sc_scatter_msg(dst3d, msg, zerosn)
    ns = _t2(xpad, agg, deg, rootpad, root_b.reshape(1, 128))
    S = sc_gcn(src3d, dst3d, ns, zerosr)
    return _t34(S, ns, deg, gcn_w, gcn_b.reshape(1, 256),
                f1w, f1b.reshape(1, 512), f2w, f2b.reshape(1, 256),
                f3w, f3b.reshape(1, 128), pw, pb.reshape(1, 7))


# speedup vs baseline: 1.3249x; 1.0040x over previous
import functools

import jax
import jax.numpy as jnp
from jax import lax
from jax.experimental import pallas as pl
from jax.experimental.pallas import tpu as pltpu
from jax.experimental.pallas import tpu_sc as plsc

N = 8192 * 6
E = 8192 * 5
NC, NS = 2, 16
NW = NC * NS
EPW = E // NW
EPT = E // NS
NPT = N // NS
IB = 128
IBS = 64
NRANGE = 4
NR = N // NRANGE
RPT = NR // NS

_SC_PARAMS = pltpu.CompilerParams(use_tc_tiling_on_sc=False)


@functools.cache
def _sc_kernels():
    mesh = plsc.VectorSubcoreMesh(
        core_axis_name="c", subcore_axis_name="s",
        num_cores=NC, num_subcores=NS)
    g0 = _make_sc_gather_deg(mesh)
    s2 = _make_sc_scatter_msg(mesh)
    s3 = _make_sc_gcn(mesh)
    return g0, s2, s3


def _make_sc_gather_deg(mesh):
    return functools.partial(
        pl.kernel,
        out_type=(jax.ShapeDtypeStruct((E, 128), jnp.float32),
                  jax.ShapeDtypeStruct((N, 128), jnp.float32)),
        mesh=mesh,
        scratch_types=(
            pltpu.VMEM((EPW // IB, IB), jnp.int32),
            pltpu.VMEM((EPT // IBS, IBS), jnp.int32),
            pltpu.VMEM((IB, 128), jnp.float32),
            pltpu.VMEM((IB, 128), jnp.float32),
            pltpu.VMEM((IBS, 8), jnp.float32),
            pltpu.VMEM_SHARED((N, 8), jnp.float32),
            pltpu.SemaphoreType.DMA,
            pltpu.SemaphoreType.DMA,
            pltpu.SemaphoreType.DMA,
            pltpu.SemaphoreType.DMA,
        ),
        compiler_params=_SC_PARAMS,
    )(_sc_gather_deg_body)


def _sc_gather_deg_body(xpad_hbm, src3d_hbm, dst3d_hbm, ones_hbm, zeros8_hbm,
                        xsrc_hbm, deg_hbm,
                        sidx_v, didx_v, rows_v0, rows_v1, ones_v, deg_sp,
                        gsem0, gsem1, wsem0, wsem1):
    cid = lax.axis_index("c")
    sid = lax.axis_index("s")
    wid = sid * NC + cid
    rows_v = (rows_v0, rows_v1)
    gsem = (gsem0, gsem1)
    wsem = (wsem0, wsem1)
    nj = EPW // IB
    pltpu.sync_copy(src3d_hbm.at[wid], sidx_v)
    gd = [None, None]
    wd = [None, None]
    for j in range(nj):
        b = j % 2
        if wd[b] is not None:
            wd[b].wait()
        gd[b] = pltpu.async_copy(xpad_hbm.at[sidx_v.at[j]], rows_v[b],
                                 gsem[b])
        if j > 0:
            pb = (j - 1) % 2
            gd[pb].wait()
            wd[pb] = pltpu.async_copy(
                rows_v[pb],
                xsrc_hbm.at[pl.ds(wid * EPW + (j - 1) * IB, IB)], wsem[pb])
    lb = (nj - 1) % 2
    gd[lb].wait()
    wd[lb] = pltpu.async_copy(
        rows_v[lb], xsrc_hbm.at[pl.ds(wid * EPW + (nj - 1) * IB, IB)],
        wsem[lb])
    wd[0].wait()
    wd[1].wait()

    @pl.when(cid == 0)
    def _deg():
        pltpu.sync_copy(zeros8_hbm, deg_sp.at[pl.ds(sid * NPT, NPT)])
        pltpu.sync_copy(dst3d_hbm.at[sid], didx_v)
        pltpu.sync_copy(ones_hbm, ones_v)
        plsc.subcore_barrier()
        descs = [pltpu.async_copy(ones_v, deg_sp.at[didx_v.at[j]], gsem0,
                                  add=True)
                 for j in range(EPT // IBS)]
        for d in descs:
            d.wait()
        plsc.subcore_barrier()
        pltpu.sync_copy(deg_sp.at[pl.ds(sid * NPT, NPT)],
                        deg_hbm.at[pl.ds(sid * NPT, NPT), pl.ds(0, 8)])


def _make_sc_scatter_msg(mesh):
    return functools.partial(
        pl.kernel,
        out_type=jax.ShapeDtypeStruct((N, 128), jnp.float32),
        mesh=mesh,
        scratch_types=(
            pltpu.VMEM((EPT // IBS, IBS), jnp.int32),
            pltpu.VMEM((IBS, 32), jnp.float32),
            pltpu.VMEM((IBS, 32), jnp.float32),
            pltpu.VMEM((IBS, 32), jnp.float32),
            pltpu.VMEM_SHARED((N, 32), jnp.float32),
            pltpu.SemaphoreType.DMA,
            pltpu.SemaphoreType.DMA,
            pltpu.SemaphoreType.DMA,
            pltpu.SemaphoreType.DMA,
            pltpu.SemaphoreType.DMA,
            pltpu.SemaphoreType.DMA,
        ),
        compiler_params=_SC_PARAMS,
    )(_sc_scatter_msg_body)


def _sc_scatter_msg_body(dst3d_hbm, msg_hbm, zeros_hbm, agg_hbm,
                         didx_v, dat_v0, dat_v1, dat_v2, acc_sp,
                         lsem0, lsem1, lsem2, ssem0, ssem1, ssem2):
    cid = lax.axis_index("c")
    sid = lax.axis_index("s")
    dat_v = (dat_v0, dat_v1, dat_v2)
    lsem = (lsem0, lsem1, lsem2)
    ssem = (ssem0, ssem1, ssem2)
    nj = EPT // IBS
    pltpu.sync_copy(dst3d_hbm.at[sid], didx_v)
    for p in range(2):
        col = cid * 64 + p * 32
        pltpu.sync_copy(zeros_hbm, acc_sp.at[pl.ds(sid * NPT, NPT)])
        plsc.subcore_barrier()
        ld = [None, None, None]
        sd = [None, None, None]
        ld[0] = pltpu.async_copy(
            msg_hbm.at[pl.ds(sid * EPT, IBS), pl.ds(col, 32)], dat_v[0],
            lsem[0])
        for j in range(nj):
            b = j % 3
            if j + 1 < nj:
                nb = (j + 1) % 3
                if sd[nb] is not None:
                    sd[nb].wait()
                ld[nb] = pltpu.async_copy(
                    msg_hbm.at[pl.ds(sid * EPT + (j + 1) * IBS, IBS),
                               pl.ds(col, 32)], dat_v[nb], lsem[nb])
            ld[b].wait()
            sd[b] = pltpu.async_copy(dat_v[b], acc_sp.at[didx_v.at[j]],
                                     ssem[b], add=True)
        for d in sd:
            if d is not None:
                d.wait()
        plsc.subcore_barrier()
        pltpu.sync_copy(acc_sp.at[pl.ds(sid * NPT, NPT)],
                        agg_hbm.at[pl.ds(sid * NPT, NPT), pl.ds(col, 32)])
        plsc.subcore_barrier()


def _rewrite_range_idx(didx_v, didx_r, lo):
    hi = lo + NR
    for r in range(EPT // IBS):
        for k in range(IBS // 16):
            v = didx_v[r, pl.ds(16 * k, 16)]
            ok = (v >= lo) & (v < hi)
            didx_r[r, pl.ds(16 * k, 16)] = jnp.where(
                ok, v - lo, jnp.full((16,), NR, jnp.int32))


def _make_sc_gcn(mesh):
    return functools.partial(
        pl.kernel,
        out_type=jax.ShapeDtypeStruct((N, 128), jnp.float32),
        mesh=mesh,
        scratch_types=(
            pltpu.VMEM((EPT // IBS, IBS), jnp.int32),
            pltpu.VMEM((EPT // IBS, IBS), jnp.int32),
            pltpu.VMEM((EPT // IBS, IBS), jnp.int32),
            pltpu.VMEM((IBS, 128), jnp.float32),
            pltpu.VMEM((IBS, 128), jnp.float32),
            pltpu.VMEM((IBS, 128), jnp.float32),
            pltpu.VMEM_SHARED((NR + 8, 128), jnp.float32),
            pltpu.SemaphoreType.DMA,
            pltpu.SemaphoreType.DMA,
            pltpu.SemaphoreType.DMA,
            pltpu.SemaphoreType.DMA,
            pltpu.SemaphoreType.DMA,
            pltpu.SemaphoreType.DMA,
        ),
        compiler_params=_SC_PARAMS,
    )(_sc_gcn_body)


def _sc_gcn_body(src3d_hbm, dst3d_hbm, ns_hbm, zeros_hbm, s_hbm,
                 sidx_v, didx_v, didx_r, dat_v0, dat_v1, dat_v2, acc_sp,
                 lsem0, lsem1, lsem2, ssem0, ssem1, ssem2):
    cid = lax.axis_index("c")
    sid = lax.axis_index("s")
    dat_v = (dat_v0, dat_v1, dat_v2)
    lsem = (lsem0, lsem1, lsem2)
    ssem = (ssem0, ssem1, ssem2)
    nj = EPT // IBS
    pltpu.sync_copy(src3d_hbm.at[sid], sidx_v)
    pltpu.sync_copy(dst3d_hbm.at[sid], didx_v)
    for p in range(2):
        lo = (2 * cid + p) * NR
        _rewrite_range_idx(didx_v, didx_r, lo)
        pltpu.sync_copy(zeros_hbm, acc_sp.at[pl.ds(sid * RPT, RPT)])
        plsc.subcore_barrier()
        ld = [None, None, None]
        sd = [None, None, None]
        ld[0] = pltpu.async_copy(ns_hbm.at[sidx_v.at[0]], dat_v[0], lsem[0])
        for j in range(nj):
            b = j % 3
            if j + 1 < nj:
                nb = (j + 1) % 3
                if sd[nb] is not None:
                    sd[nb].wait()
                ld[nb] = pltpu.async_copy(ns_hbm.at[sidx_v.at[j + 1]],
                                          dat_v[nb], lsem[nb])
            ld[b].wait()
            sd[b] = pltpu.async_copy(dat_v[b], acc_sp.at[didx_r.at[j]],
                                     ssem[b], add=True)
        for d in sd:
            if d is not None:
                d.wait()
        plsc.subcore_barrier()
        pltpu.sync_copy(acc_sp.at[pl.ds(sid * RPT, RPT)],
                        s_hbm.at[pl.ds(lo + sid * RPT, RPT)])
        plsc.subcore_barrier()


EB = 2048


def _t1_body(ea_ref, xs_ref, w1_ref, b1_ref, w2_ref, b2_ref, o_ref):
    bf16 = jnp.bfloat16
    h1 = jnp.maximum(
        jnp.dot(ea_ref[...], w1_ref[...],
                preferred_element_type=jnp.float32) + b1_ref[...], 0.0)
    h1b = h1.astype(bf16)
    w2b = w2_ref[...].astype(bf16)
    xs = xs_ref[...]
    msg = jnp.zeros((EB, 128), jnp.float32)
    for i in range(7):
        h2i = jnp.maximum(
            jnp.dot(h1b, w2b[:, 128 * i:128 * (i + 1)],
                    preferred_element_type=jnp.float32)
            + b2_ref[:, 128 * i:128 * (i + 1)], 0.0)
        msg = msg + xs[:, i:i + 1] * h2i
    o_ref[...] = msg


def _t1(ea, xsrc, W1, b1, W2, b2):
    return pl.pallas_call(
        _t1_body,
        grid=(E // EB,),
        in_specs=[
            pl.BlockSpec((EB, 19), lambda e: (e, 0)),
            pl.BlockSpec((EB, 128), lambda e: (e, 0)),
            pl.BlockSpec((19, 128), lambda e: (0, 0)),
            pl.BlockSpec((1, 128), lambda e: (0, 0)),
            pl.BlockSpec((128, 896), lambda e: (0, 0)),
            pl.BlockSpec((1, 896), lambda e: (0, 0)),
        ],
        out_specs=pl.BlockSpec((EB, 128), lambda e: (e, 0)),
        out_shape=jax.ShapeDtypeStruct((E, 128), jnp.float32),
    )(ea, xsrc, W1, b1, W2, b2)


NB = 4096


def _t2_body(xp_ref, agg_ref, deg_ref, rw_ref, rb_ref, ns_ref):
    node = (jnp.dot(xp_ref[...], rw_ref[...],
                    preferred_element_type=jnp.float32)
            + rb_ref[...] + agg_ref[...])
    dinv = lax.rsqrt(deg_ref[:, 0:1] + 1.0)
    ns_ref[...] = dinv * node


def _t2(xpad, agg, deg, rootpad, root_b):
    f32 = jnp.float32
    return pl.pallas_call(
        _t2_body,
        grid=(N // NB,),
        in_specs=[
            pl.BlockSpec((NB, 128), lambda n: (n, 0)),
            pl.BlockSpec((NB, 128), lambda n: (n, 0)),
            pl.BlockSpec((NB, 128), lambda n: (n, 0)),
            pl.BlockSpec((128, 128), lambda n: (0, 0)),
            pl.BlockSpec((1, 128), lambda n: (0, 0)),
        ],
        out_specs=pl.BlockSpec((NB, 128), lambda n: (n, 0)),
        out_shape=jax.ShapeDtypeStruct((N, 128), f32),
    )(xpad, agg, deg, rootpad, root_b)


GB = 512


def _t34_body(s_ref, ns_ref, deg_ref, gw_ref, gb_ref,
              w1, b1, w2, b2, w3, b3, pw, pb, out_ref):
    bf16 = jnp.bfloat16
    dinv = lax.rsqrt(deg_ref[:, 0:1] + 1.0)
    pre = dinv * (s_ref[...] + ns_ref[...])
    h2 = jnp.maximum(
        jnp.dot(pre, gw_ref[...], preferred_element_type=jnp.float32)
        + gb_ref[...], 0.0)
    h23 = h2.reshape(GB, 6, 256).astype(bf16)
    w1b = w1[...].astype(bf16)
    t = jnp.zeros((GB, 512), jnp.float32)
    for j in range(6):
        t = t + jnp.dot(h23[:, j, :], w1b[256 * j:256 * (j + 1), :],
                        preferred_element_type=jnp.float32)
    t = jnp.maximum(t + b1[...], 0.0)
    t = jnp.maximum(
        jnp.dot(t.astype(bf16), w2[...].astype(bf16),
                preferred_element_type=jnp.float32) + b2[...], 0.0)
    t = jnp.maximum(
        jnp.dot(t.astype(bf16), w3[...].astype(bf16),
                preferred_element_type=jnp.float32) + b3[...], 0.0)
    out_ref[...] = (jnp.dot(t, pw[...], preferred_element_type=jnp.float32)
                    + pb[...])


def _t34(S, ns, deg, gcn_w, gcn_b, f1w, f1b, f2w, f2b, f3w, f3b, pw, pb):
    NG = N // 6
    NBR = 6 * GB
    return pl.pallas_call(
        _t34_body,
        grid=(NG // GB,),
        in_specs=[
            pl.BlockSpec((NBR, 128), lambda n: (n, 0)),
            pl.BlockSpec((NBR, 128), lambda n: (n, 0)),
            pl.BlockSpec((NBR, 128), lambda n: (n, 0)),
            pl.BlockSpec((128, 256), lambda n: (0, 0)),
            pl.BlockSpec((1, 256), lambda n: (0, 0)),
            pl.BlockSpec((1536, 512), lambda n: (0, 0)),
            pl.BlockSpec((1, 512), lambda n: (0, 0)),
            pl.BlockSpec((512, 256), lambda n: (0, 0)),
            pl.BlockSpec((1, 256), lambda n: (0, 0)),
            pl.BlockSpec((256, 128), lambda n: (0, 0)),
            pl.BlockSpec((1, 128), lambda n: (0, 0)),
            pl.BlockSpec((128, 7), lambda n: (0, 0)),
            pl.BlockSpec((1, 7), lambda n: (0, 0)),
        ],
        out_specs=pl.BlockSpec((GB, 7), lambda n: (n, 0)),
        out_shape=jax.ShapeDtypeStruct((NG, 7), jnp.float32),
    )(S, ns, deg, gcn_w, gcn_b, f1w, f1b, f2w, f2b, f3w, f3b, pw, pb)


def kernel(x, edge_index, edge_attr, num_graphs, W1, b1, W2, b2, root,
           root_b, gcn_w, gcn_b, f1w, f1b, f2w, f2b, f3w, f3b, pw, pb):
    f32 = jnp.float32
    src = edge_index[0].astype(jnp.int32)
    dst = edge_index[1].astype(jnp.int32)
    src3d_w = src.reshape(NW, EPW // IB, IB)
    src3d = src.reshape(NS, EPT // IBS, IBS)
    dst3d = dst.reshape(NS, EPT // IBS, IBS)
    xpad = jnp.pad(x, ((0, 0), (0, 121)))
    rootpad = jnp.pad(root, ((0, 121), (0, 0)))
    ones8 = jnp.ones((IBS, 8), f32)
    zeros8 = jnp.zeros((NPT, 8), f32)
    zerosn = jnp.zeros((NPT, 32), f32)
    zerosr = jnp.zeros((RPT, 128), f32)

    sc_gather_deg, sc_scatter_msg, sc_gcn = _sc_kernels()
    xsrc, deg = sc_gather_deg(xpad, src3d_w, dst3d, ones8, zeros8)
    msg = _t1(edge_attr, xsrc, W1, b1.reshape(1, 128), W2,
              b2.reshape(1, 896))
    agg = sc_scatter_msg(dst3d, msg, zerosn)
    ns = _t2(xpad, agg, deg, rootpad, root_b.reshape(1, 128))
    S = sc_gcn(src3d, dst3d, ns, zerosr)
    return _t34(S, ns, deg, gcn_w, gcn_b.reshape(1, 256),
                f1w, f1b.reshape(1, 512), f2w, f2b.reshape(1, 256),
                f3w, f3b.reshape(1, 128), pw, pb.reshape(1, 7))


# speedup vs baseline: 1.3571x; 1.0243x over previous
import functools

import jax
import jax.numpy as jnp
from jax import lax
from jax.experimental import pallas as pl
from jax.experimental.pallas import tpu as pltpu
from jax.experimental.pallas import tpu_sc as plsc

N = 8192 * 6
E = 8192 * 5
NC, NS = 2, 16
NW = NC * NS
EPW = E // NW
EPT = E // NS
NPT = N // NS
IB = 128
IBS = 64
IB2 = 128
NRANGE = 4
NR = N // NRANGE
RPT = NR // NS

_SC_PARAMS = pltpu.CompilerParams(use_tc_tiling_on_sc=False)


@functools.cache
def _sc_kernels():
    mesh = plsc.VectorSubcoreMesh(
        core_axis_name="c", subcore_axis_name="s",
        num_cores=NC, num_subcores=NS)
    g0 = _make_sc_gather_deg(mesh)
    s2 = _make_sc_scatter_msg(mesh)
    s3 = _make_sc_gcn(mesh)
    return g0, s2, s3


def _make_sc_gather_deg(mesh):
    return functools.partial(
        pl.kernel,
        out_type=(jax.ShapeDtypeStruct((E, 128), jnp.float32),
                  jax.ShapeDtypeStruct((N, 128), jnp.float32)),
        mesh=mesh,
        scratch_types=(
            pltpu.VMEM((EPW // IB, IB), jnp.int32),
            pltpu.VMEM((EPT // IBS, IBS), jnp.int32),
            pltpu.VMEM((IB, 128), jnp.float32),
            pltpu.VMEM((IB, 128), jnp.float32),
            pltpu.VMEM((IBS, 8), jnp.float32),
            pltpu.VMEM_SHARED((N, 8), jnp.float32),
            pltpu.SemaphoreType.DMA,
            pltpu.SemaphoreType.DMA,
            pltpu.SemaphoreType.DMA,
            pltpu.SemaphoreType.DMA,
        ),
        compiler_params=_SC_PARAMS,
    )(_sc_gather_deg_body)


def _sc_gather_deg_body(xpad_hbm, src3d_hbm, dst3d_hbm, ones_hbm, zeros8_hbm,
                        xsrc_hbm, deg_hbm,
                        sidx_v, didx_v, rows_v0, rows_v1, ones_v, deg_sp,
                        gsem0, gsem1, wsem0, wsem1):
    cid = lax.axis_index("c")
    sid = lax.axis_index("s")
    wid = sid * NC + cid
    rows_v = (rows_v0, rows_v1)
    gsem = (gsem0, gsem1)
    wsem = (wsem0, wsem1)
    nj = EPW // IB
    pltpu.sync_copy(src3d_hbm.at[wid], sidx_v)
    gd = [None, None]
    wd = [None, None]
    for j in range(nj):
        b = j % 2
        if wd[b] is not None:
            wd[b].wait()
        gd[b] = pltpu.async_copy(xpad_hbm.at[sidx_v.at[j]], rows_v[b],
                                 gsem[b])
        if j > 0:
            pb = (j - 1) % 2
            gd[pb].wait()
            wd[pb] = pltpu.async_copy(
                rows_v[pb],
                xsrc_hbm.at[pl.ds(wid * EPW + (j - 1) * IB, IB)], wsem[pb])
    lb = (nj - 1) % 2
    gd[lb].wait()
    wd[lb] = pltpu.async_copy(
        rows_v[lb], xsrc_hbm.at[pl.ds(wid * EPW + (nj - 1) * IB, IB)],
        wsem[lb])
    wd[0].wait()
    wd[1].wait()

    @pl.when(cid == 0)
    def _deg():
        pltpu.sync_copy(zeros8_hbm, deg_sp.at[pl.ds(sid * NPT, NPT)])
        pltpu.sync_copy(dst3d_hbm.at[sid], didx_v)
        pltpu.sync_copy(ones_hbm, ones_v)
        plsc.subcore_barrier()
        descs = [pltpu.async_copy(ones_v, deg_sp.at[didx_v.at[j]], gsem0,
                                  add=True)
                 for j in range(EPT // IBS)]
        for d in descs:
            d.wait()
        plsc.subcore_barrier()
        pltpu.sync_copy(deg_sp.at[pl.ds(sid * NPT, NPT)],
                        deg_hbm.at[pl.ds(sid * NPT, NPT), pl.ds(0, 8)])


def _make_sc_scatter_msg(mesh):
    return functools.partial(
        pl.kernel,
        out_type=jax.ShapeDtypeStruct((N, 128), jnp.float32),
        mesh=mesh,
        scratch_types=(
            pltpu.VMEM((EPT // IB2, IB2), jnp.int32),
            pltpu.VMEM((IB2, 32), jnp.float32),
            pltpu.VMEM((IB2, 32), jnp.float32),
            pltpu.VMEM((IB2, 32), jnp.float32),
            pltpu.VMEM_SHARED((N, 32), jnp.float32),
            pltpu.SemaphoreType.DMA,
            pltpu.SemaphoreType.DMA,
            pltpu.SemaphoreType.DMA,
            pltpu.SemaphoreType.DMA,
            pltpu.SemaphoreType.DMA,
            pltpu.SemaphoreType.DMA,
        ),
        compiler_params=_SC_PARAMS,
    )(_sc_scatter_msg_body)


def _sc_scatter_msg_body(dst3d_hbm, msg_hbm, zeros_hbm, agg_hbm,
                         didx_v, dat_v0, dat_v1, dat_v2, acc_sp,
                         lsem0, lsem1, lsem2, ssem0, ssem1, ssem2):
    cid = lax.axis_index("c")
    sid = lax.axis_index("s")
    dat_v = (dat_v0, dat_v1, dat_v2)
    lsem = (lsem0, lsem1, lsem2)
    ssem = (ssem0, ssem1, ssem2)
    nj = EPT // IB2
    pltpu.sync_copy(dst3d_hbm.at[sid], didx_v)
    for p in range(2):
        col = cid * 64 + p * 32
        pltpu.sync_copy(zeros_hbm, acc_sp.at[pl.ds(sid * NPT, NPT)])
        plsc.subcore_barrier()
        ld = [None, None, None]
        sd = [None, None, None]
        ld[0] = pltpu.async_copy(
            msg_hbm.at[pl.ds(sid * EPT, IB2), pl.ds(col, 32)], dat_v[0],
            lsem[0])
        for j in range(nj):
            b = j % 3
            if j + 1 < nj:
                nb = (j + 1) % 3
                if sd[nb] is not None:
                    sd[nb].wait()
                ld[nb] = pltpu.async_copy(
                    msg_hbm.at[pl.ds(sid * EPT + (j + 1) * IB2, IB2),
                               pl.ds(col, 32)], dat_v[nb], lsem[nb])
            ld[b].wait()
            sd[b] = pltpu.async_copy(dat_v[b], acc_sp.at[didx_v.at[j]],
                                     ssem[b], add=True)
        for d in sd:
            if d is not None:
                d.wait()
        plsc.subcore_barrier()
        pltpu.sync_copy(acc_sp.at[pl.ds(sid * NPT, NPT)],
                        agg_hbm.at[pl.ds(sid * NPT, NPT), pl.ds(col, 32)])
        plsc.subcore_barrier()


def _rewrite_range_idx(didx_v, didx_r, lo):
    hi = lo + NR
    for r in range(EPT // IBS):
        for k in range(IBS // 16):
            v = didx_v[r, pl.ds(16 * k, 16)]
            ok = (v >= lo) & (v < hi)
            didx_r[r, pl.ds(16 * k, 16)] = jnp.where(
                ok, v - lo, jnp.full((16,), NR, jnp.int32))


def _make_sc_gcn(mesh):
    return functools.partial(
        pl.kernel,
        out_type=jax.ShapeDtypeStruct((N, 128), jnp.float32),
        mesh=mesh,
        scratch_types=(
            pltpu.VMEM((EPT // IBS, IBS), jnp.int32),
            pltpu.VMEM((EPT // IBS, IBS), jnp.int32),
            pltpu.VMEM((EPT // IBS, IBS), jnp.int32),
            pltpu.VMEM((IBS, 128), jnp.float32),
            pltpu.VMEM((IBS, 128), jnp.float32),
            pltpu.VMEM((IBS, 128), jnp.float32),
            pltpu.VMEM_SHARED((NR + 8, 128), jnp.float32),
            pltpu.SemaphoreType.DMA,
            pltpu.SemaphoreType.DMA,
            pltpu.SemaphoreType.DMA,
            pltpu.SemaphoreType.DMA,
            pltpu.SemaphoreType.DMA,
            pltpu.SemaphoreType.DMA,
        ),
        compiler_params=_SC_PARAMS,
    )(_sc_gcn_body)


def _sc_gcn_body(src3d_hbm, dst3d_hbm, ns_hbm, zeros_hbm, s_hbm,
                 sidx_v, didx_v, didx_r, dat_v0, dat_v1, dat_v2, acc_sp,
                 lsem0, lsem1, lsem2, ssem0, ssem1, ssem2):
    cid = lax.axis_index("c")
    sid = lax.axis_index("s")
    dat_v = (dat_v0, dat_v1, dat_v2)
    lsem = (lsem0, lsem1, lsem2)
    ssem = (ssem0, ssem1, ssem2)
    nj = EPT // IBS
    pltpu.sync_copy(src3d_hbm.at[sid], sidx_v)
    pltpu.sync_copy(dst3d_hbm.at[sid], didx_v)
    for p in range(2):
        lo = (2 * cid + p) * NR
        _rewrite_range_idx(didx_v, didx_r, lo)
        pltpu.sync_copy(zeros_hbm, acc_sp.at[pl.ds(sid * RPT, RPT)])
        plsc.subcore_barrier()
        ld = [None, None, None]
        sd = [None, None, None]
        ld[0] = pltpu.async_copy(ns_hbm.at[sidx_v.at[0]], dat_v[0], lsem[0])
        for j in range(nj):
            b = j % 3
            if j + 1 < nj:
                nb = (j + 1) % 3
                if sd[nb] is not None:
                    sd[nb].wait()
                ld[nb] = pltpu.async_copy(ns_hbm.at[sidx_v.at[j + 1]],
                                          dat_v[nb], lsem[nb])
            ld[b].wait()
            sd[b] = pltpu.async_copy(dat_v[b], acc_sp.at[didx_r.at[j]],
                                     ssem[b], add=True)
        for d in sd:
            if d is not None:
                d.wait()
        plsc.subcore_barrier()
        pltpu.sync_copy(acc_sp.at[pl.ds(sid * RPT, RPT)],
                        s_hbm.at[pl.ds(lo + sid * RPT, RPT)])
        plsc.subcore_barrier()


EB = 2048


def _t1_body(ea_ref, xs_ref, w1_ref, b1_ref, w2_ref, b2_ref, o_ref):
    bf16 = jnp.bfloat16
    h1 = jnp.maximum(
        jnp.dot(ea_ref[...], w1_ref[...],
                preferred_element_type=jnp.float32) + b1_ref[...], 0.0)
    h1b = h1.astype(bf16)
    w2b = w2_ref[...].astype(bf16)
    xs = xs_ref[...]
    msg = jnp.zeros((EB, 128), jnp.float32)
    for i in range(7):
        h2i = jnp.maximum(
            jnp.dot(h1b, w2b[:, 128 * i:128 * (i + 1)],
                    preferred_element_type=jnp.float32)
            + b2_ref[:, 128 * i:128 * (i + 1)], 0.0)
        msg = msg + xs[:, i:i + 1] * h2i
    o_ref[...] = msg


def _t1(ea, xsrc, W1, b1, W2, b2):
    return pl.pallas_call(
        _t1_body,
        grid=(E // EB,),
        in_specs=[
            pl.BlockSpec((EB, 19), lambda e: (e, 0)),
            pl.BlockSpec((EB, 128), lambda e: (e, 0)),
            pl.BlockSpec((19, 128), lambda e: (0, 0)),
            pl.BlockSpec((1, 128), lambda e: (0, 0)),
            pl.BlockSpec((128, 896), lambda e: (0, 0)),
            pl.BlockSpec((1, 896), lambda e: (0, 0)),
        ],
        out_specs=pl.BlockSpec((EB, 128), lambda e: (e, 0)),
        out_shape=jax.ShapeDtypeStruct((E, 128), jnp.float32),
    )(ea, xsrc, W1, b1, W2, b2)


NB = 4096


def _t2_body(xp_ref, agg_ref, deg_ref, rw_ref, rb_ref, ns_ref):
    node = (jnp.dot(xp_ref[...], rw_ref[...],
                    preferred_element_type=jnp.float32)
            + rb_ref[...] + agg_ref[...])
    dinv = lax.rsqrt(deg_ref[:, 0:1] + 1.0)
    ns_ref[...] = dinv * node


def _t2(xpad, agg, deg, rootpad, root_b):
    f32 = jnp.float32
    return pl.pallas_call(
        _t2_body,
        grid=(N // NB,),
        in_specs=[
            pl.BlockSpec((NB, 128), lambda n: (n, 0)),
            pl.BlockSpec((NB, 128), lambda n: (n, 0)),
            pl.BlockSpec((NB, 128), lambda n: (n, 0)),
            pl.BlockSpec((128, 128), lambda n: (0, 0)),
            pl.BlockSpec((1, 128), lambda n: (0, 0)),
        ],
        out_specs=pl.BlockSpec((NB, 128), lambda n: (n, 0)),
        out_shape=jax.ShapeDtypeStruct((N, 128), f32),
    )(xpad, agg, deg, rootpad, root_b)


GB = 512


def _t34_body(s_ref, ns_ref, deg_ref, gw_ref, gb_ref,
              w1, b1, w2, b2, w3, b3, pw, pb, out_ref):
    bf16 = jnp.bfloat16
    dinv = lax.rsqrt(deg_ref[:, 0:1] + 1.0)
    pre = dinv * (s_ref[...] + ns_ref[...])
    h2 = jnp.maximum(
        jnp.dot(pre, gw_ref[...], preferred_element_type=jnp.float32)
        + gb_ref[...], 0.0)
    h23 = h2.reshape(GB, 6, 256).astype(bf16)
    w1b = w1[...].astype(bf16)
    t = jnp.zeros((GB, 512), jnp.float32)
    for j in range(6):
        t = t + jnp.dot(h23[:, j, :], w1b[256 * j:256 * (j + 1), :],
                        preferred_element_type=jnp.float32)
    t = jnp.maximum(t + b1[...], 0.0)
    t = jnp.maximum(
        jnp.dot(t.astype(bf16), w2[...].astype(bf16),
                preferred_element_type=jnp.float32) + b2[...], 0.0)
    t = jnp.maximum(
        jnp.dot(t.astype(bf16), w3[...].astype(bf16),
                preferred_element_type=jnp.float32) + b3[...], 0.0)
    out_ref[...] = (jnp.dot(t, pw[...], preferred_element_type=jnp.float32)
                    + pb[...])


def _t34(S, ns, deg, gcn_w, gcn_b, f1w, f1b, f2w, f2b, f3w, f3b, pw, pb):
    NG = N // 6
    NBR = 6 * GB
    return pl.pallas_call(
        _t34_body,
        grid=(NG // GB,),
        in_specs=[
            pl.BlockSpec((NBR, 128), lambda n: (n, 0)),
            pl.BlockSpec((NBR, 128), lambda n: (n, 0)),
            pl.BlockSpec((NBR, 128), lambda n: (n, 0)),
            pl.BlockSpec((128, 256), lambda n: (0, 0)),
            pl.BlockSpec((1, 256), lambda n: (0, 0)),
            pl.BlockSpec((1536, 512), lambda n: (0, 0)),
            pl.BlockSpec((1, 512), lambda n: (0, 0)),
            pl.BlockSpec((512, 256), lambda n: (0, 0)),
            pl.BlockSpec((1, 256), lambda n: (0, 0)),
            pl.BlockSpec((256, 128), lambda n: (0, 0)),
            pl.BlockSpec((1, 128), lambda n: (0, 0)),
            pl.BlockSpec((128, 7), lambda n: (0, 0)),
            pl.BlockSpec((1, 7), lambda n: (0, 0)),
        ],
        out_specs=pl.BlockSpec((GB, 7), lambda n: (n, 0)),
        out_shape=jax.ShapeDtypeStruct((NG, 7), jnp.float32),
    )(S, ns, deg, gcn_w, gcn_b, f1w, f1b, f2w, f2b, f3w, f3b, pw, pb)


def kernel(x, edge_index, edge_attr, num_graphs, W1, b1, W2, b2, root,
           root_b, gcn_w, gcn_b, f1w, f1b, f2w, f2b, f3w, f3b, pw, pb):
    f32 = jnp.float32
    src = edge_index[0].astype(jnp.int32)
    dst = edge_index[1].astype(jnp.int32)
    src3d_w = src.reshape(NW, EPW // IB, IB)
    src3d = src.reshape(NS, EPT // IBS, IBS)
    dst3d = dst.reshape(NS, EPT // IBS, IBS)
    dst3d2 = dst.reshape(NS, EPT // IB2, IB2)
    xpad = jnp.pad(x, ((0, 0), (0, 121)))
    rootpad = jnp.pad(root, ((0, 121), (0, 0)))
    ones8 = jnp.ones((IBS, 8), f32)
    zeros8 = jnp.zeros((NPT, 8), f32)
    zerosn = jnp.zeros((NPT, 32), f32)
    zerosr = jnp.zeros((RPT, 128), f32)

    sc_gather_deg, sc_scatter_msg, sc_gcn = _sc_kernels()
    xsrc, deg = sc_gather_deg(xpad, src3d_w, dst3d, ones8, zeros8)
    msg = _t1(edge_attr, xsrc, W1, b1.reshape(1, 128), W2,
              b2.reshape(1, 896))
    agg = sc_scatter_msg(dst3d2, msg, zerosn)
    ns = _t2(xpad, agg, deg, rootpad, root_b.reshape(1, 128))
    S = sc_gcn(src3d, dst3d, ns, zerosr)
    return _t34(S, ns, deg, gcn_w, gcn_b.reshape(1, 256),
                f1w, f1b.reshape(1, 512), f2w, f2b.reshape(1, 256),
                f3w, f3b.reshape(1, 128), pw, pb.reshape(1, 7))
